# Initial kernel scaffold; baseline (speedup 1.0000x reference)
import jax
import jax.numpy as jnp
from jax.experimental import pallas as pl


def kernel(x, edge_index, node_time, seed_time, batch_idx, W_enc, b_enc, Wt, bt, Ws0, Wn0, b0, Ws1, Wn1, b1, Wh, bh):
    raise NotImplementedError("write your pallas kernel here")


# speedup vs baseline: 8.8309x; 8.8309x over previous
import functools

import numpy as np
import jax
import jax.numpy as jnp
from jax import lax
from jax.experimental import pallas as pl
from jax.experimental.pallas import tpu as pltpu
from jax.experimental.pallas import tpu_sc as plsc

_N = 10000
_E = 320000
_C = 128
_S = 1024
_NP = 10240

_NC = 2
_NS = 16
_NW = _NC * _NS

_REL_PER_W = _NP // _NW
_E_PER_SC_TILE = _E // _NS
_E_PER_W = _E // _NW
_B = 80
_NCHUNK = _E_PER_W // _B
_ROWS_PER_TILE = _NP // _NS

def _k0_body(dst_hbm, nt_hbm, st_hbm, bi_hbm, rel_out, rdeg_out,
             st_v, bi_v, nt_v, rel_v, deg_v, dstbuf, dslice_v, dtmp_v,
             rdeg_v, deg_sh):
    cid = lax.axis_index("c")
    sid = lax.axis_index("s")
    wid = sid * _NC + cid

    pltpu.sync_copy(st_hbm, st_v)
    pltpu.sync_copy(bi_hbm.at[pl.ds(wid * _REL_PER_W, _REL_PER_W)], bi_v)
    pltpu.sync_copy(nt_hbm.at[pl.ds(wid * _REL_PER_W, _REL_PER_W)], nt_v)

    def rel_step(i, _):
        b16 = bi_v[pl.ds(i * 16, 16)]
        st16 = plsc.load_gather(st_v, [b16])
        nt16 = nt_v[pl.ds(i * 16, 16)]
        rel_v[pl.ds(i * 16, 16)] = (st16 - nt16).astype(jnp.float32) / 86400.0
        return _

    lax.fori_loop(0, _REL_PER_W // 16, rel_step, None)
    pltpu.sync_copy(rel_v, rel_out.at[pl.ds(wid * _REL_PER_W, _REL_PER_W)])

    zeros16 = jnp.zeros((16,), jnp.float32)

    def z_step(i, _):
        deg_v[pl.ds(i * 16, 16)] = zeros16
        return _

    lax.fori_loop(0, _NP // 16, z_step, None)

    pltpu.sync_copy(dst_hbm.at[pl.ds(sid * _E_PER_SC_TILE, _E_PER_SC_TILE)], dstbuf)
    ones16 = jnp.full((16,), 1.0, jnp.float32)

    def d_step(i, _):
        d16 = dstbuf[pl.ds(i * 16, 16)]
        plsc.addupdate_scatter(deg_v, [d16], ones16)
        return _

    lax.fori_loop(0, _E_PER_SC_TILE // 16, d_step, None)
    pltpu.sync_copy(deg_v, deg_sh.at[sid])
    plsc.subcore_barrier()

    r0 = sid * _ROWS_PER_TILE
    pltpu.sync_copy(deg_sh.at[0, pl.ds(r0, _ROWS_PER_TILE)], dslice_v)

    def c_step(j, _):
        pltpu.sync_copy(deg_sh.at[j, pl.ds(r0, _ROWS_PER_TILE)], dtmp_v)

        def a_step(i, __):
            dslice_v[pl.ds(i * 16, 16)] += dtmp_v[pl.ds(i * 16, 16)]
            return __

        lax.fori_loop(0, _ROWS_PER_TILE // 16, a_step, None)
        return _

    lax.fori_loop(1, _NS, c_step, None)

    def r_step(i, _):
        dd = dslice_v[pl.ds(i * 16, 16)]
        rdeg_v[pl.ds(i * 16, 16)] = 1.0 / jnp.maximum(dd, 1.0)
        return _

    lax.fori_loop(0, _ROWS_PER_TILE // 16, r_step, None)

    @pl.when(cid == 0)
    def _write_rdeg():
        pltpu.sync_copy(rdeg_v, rdeg_out.at[pl.ds(sid * _ROWS_PER_TILE, _ROWS_PER_TILE)])


@functools.cache
def _get_k0():
  return functools.partial(
    pl.kernel,
    mesh=plsc.VectorSubcoreMesh(core_axis_name="c", subcore_axis_name="s"),
    compiler_params=pltpu.CompilerParams(needs_layout_passes=False),
    out_type=(
        jax.ShapeDtypeStruct((_NP,), jnp.float32),
        jax.ShapeDtypeStruct((_NP,), jnp.float32),
    ),
    scratch_types=[
        pltpu.VMEM((_S,), jnp.int32),
        pltpu.VMEM((_REL_PER_W,), jnp.int32),
        pltpu.VMEM((_REL_PER_W,), jnp.int32),
        pltpu.VMEM((_REL_PER_W,), jnp.float32),
        pltpu.VMEM((_NP,), jnp.float32),
        pltpu.VMEM((_E_PER_SC_TILE,), jnp.int32),
        pltpu.VMEM((_ROWS_PER_TILE,), jnp.float32),
        pltpu.VMEM((_ROWS_PER_TILE,), jnp.float32),
        pltpu.VMEM((_ROWS_PER_TILE,), jnp.float32),
        pltpu.VMEM_SHARED((_NS, _NP), jnp.float32),
    ],
  )(_k0_body)


def _sc2_body(h0_hbm, src_hbm, dst_hbm, out_hbm,
              src_v, dst_v, rows_v, zbuf, acc_sh, sem):
    cid = lax.axis_index("c")
    sid = lax.axis_index("s")
    wid = sid * _NC + cid

    zeros16 = jnp.zeros((16,), jnp.float32)

    def zb_step(r, _):
        for j in range(_C // 16):
            zbuf[r, pl.ds(j * 16, 16)] = zeros16
        return _

    lax.fori_loop(0, _B, zb_step, None)
    for j in range(_ROWS_PER_TILE // _B):
        pltpu.sync_copy(zbuf, acc_sh.at[pl.ds(sid * _ROWS_PER_TILE + j * _B, _B)])
    plsc.subcore_barrier()

    base = wid * _E_PER_W

    def step(i, _):
        off = base + i * _B
        pltpu.sync_copy(src_hbm.at[pl.ds(off, _B)], src_v)
        pltpu.sync_copy(dst_hbm.at[pl.ds(off, _B)], dst_v)
        pltpu.async_copy(h0_hbm.at[src_v], rows_v, sem).wait()
        pltpu.sync_copy(rows_v, acc_sh.at[dst_v], add=True)
        return _

    lax.fori_loop(0, _NCHUNK, step, None)
    plsc.subcore_barrier()

    def wb_step(j, _):
        r0 = sid * _ROWS_PER_TILE + j * _B
        pltpu.sync_copy(acc_sh.at[pl.ds(r0, _B)], rows_v)
        pltpu.sync_copy(rows_v, out_hbm.at[cid, pl.ds(r0, _B)])
        return _

    lax.fori_loop(0, _ROWS_PER_TILE // _B, wb_step, None)


@functools.cache
def _get_sc2():
  return functools.partial(
    pl.kernel,
    mesh=plsc.VectorSubcoreMesh(core_axis_name="c", subcore_axis_name="s"),
    compiler_params=pltpu.CompilerParams(needs_layout_passes=False),
    out_type=jax.ShapeDtypeStruct((_NC, _NP, _C), jnp.float32),
    scratch_types=[
        pltpu.VMEM((_B,), jnp.int32),
        pltpu.VMEM((_B,), jnp.int32),
        pltpu.VMEM((_B, _C), jnp.float32),
        pltpu.VMEM((_B, _C), jnp.float32),
        pltpu.VMEM_SHARED((_NP, _C), jnp.float32),
        pltpu.SemaphoreType.DMA,
    ],
  )(_sc2_body)


def _sc4_body(z_hbm, src_hbm, dst_hbm, hu_hbm, rdeg_hbm, out_hbm,
              z_v, src_v, dst_v, zacc_v, hu_v, rd_v, zslice_v, ztmp_v, o_v,
              zs_sh):
    cid = lax.axis_index("c")
    sid = lax.axis_index("s")

    zeros16 = jnp.zeros((16,), jnp.float32)

    def z_step(i, _):
        zacc_v[pl.ds(i * 16, 16)] = zeros16
        return _

    lax.fori_loop(0, _S // 16, z_step, None)

    pltpu.sync_copy(z_hbm, z_v)
    pltpu.sync_copy(src_hbm.at[pl.ds(sid * _E_PER_SC_TILE, _E_PER_SC_TILE)], src_v)
    pltpu.sync_copy(dst_hbm.at[pl.ds(sid * _E_PER_SC_TILE, _E_PER_SC_TILE)], dst_v)

    def e_step(i, _):
        s16 = src_v[pl.ds(i * 16, 16)]
        d16 = dst_v[pl.ds(i * 16, 16)]
        zz = plsc.load_gather(z_v, [s16])
        m = d16 < _S
        dsafe = jnp.where(m, d16, 0)
        plsc.addupdate_scatter(zacc_v, [dsafe], zz, mask=m)
        return _

    lax.fori_loop(0, _E_PER_SC_TILE // 16, e_step, None)
    pltpu.sync_copy(zacc_v, zs_sh.at[sid])
    plsc.subcore_barrier()

    spw = _S // _NS
    pltpu.sync_copy(zs_sh.at[0, pl.ds(sid * spw, spw)], zslice_v)

    def zc_step(j, _):
        pltpu.sync_copy(zs_sh.at[j, pl.ds(sid * spw, spw)], ztmp_v)

        def za_step(i, __):
            zslice_v[pl.ds(i * 16, 16)] += ztmp_v[pl.ds(i * 16, 16)]
            return __

        lax.fori_loop(0, spw // 16, za_step, None)
        return _

    lax.fori_loop(1, _NS, zc_step, None)
    pltpu.sync_copy(hu_hbm.at[pl.ds(sid * spw, spw)], hu_v)
    pltpu.sync_copy(rdeg_hbm.at[pl.ds(sid * spw, spw)], rd_v)

    def f_step(i, _):
        o_v[pl.ds(i * 16, 16)] = (hu_v[pl.ds(i * 16, 16)]
                                  + zslice_v[pl.ds(i * 16, 16)]
                                  * rd_v[pl.ds(i * 16, 16)])
        return _

    lax.fori_loop(0, spw // 16, f_step, None)

    @pl.when(cid == 0)
    def _write_out():
        pltpu.sync_copy(o_v, out_hbm.at[pl.ds(sid * spw, spw)])


@functools.cache
def _get_sc4():
  return functools.partial(
    pl.kernel,
    mesh=plsc.VectorSubcoreMesh(core_axis_name="c", subcore_axis_name="s"),
    compiler_params=pltpu.CompilerParams(needs_layout_passes=False),
    out_type=jax.ShapeDtypeStruct((_S,), jnp.float32),
    scratch_types=[
        pltpu.VMEM((_NP,), jnp.float32),
        pltpu.VMEM((_E_PER_SC_TILE,), jnp.int32),
        pltpu.VMEM((_E_PER_SC_TILE,), jnp.int32),
        pltpu.VMEM((_S,), jnp.float32),
        pltpu.VMEM((_S // _NS,), jnp.float32),
        pltpu.VMEM((_S // _NS,), jnp.float32),
        pltpu.VMEM((_S // _NS,), jnp.float32),
        pltpu.VMEM((_S // _NS,), jnp.float32),
        pltpu.VMEM((_S // _NS,), jnp.float32),
        pltpu.VMEM_SHARED((_NS, _S), jnp.float32),
    ],
  )(_sc4_body)


_BLK = 2048


def _tc1_body(x_ref, rel_ref, div_ref, we_ref, wte_ref, wto_ref, b_ref, o_ref):
    ang = rel_ref[...] * div_ref[...]
    h = jnp.dot(x_ref[...], we_ref[...], preferred_element_type=jnp.float32)
    h = h + jnp.dot(jnp.sin(ang), wte_ref[...], preferred_element_type=jnp.float32)
    h = h + jnp.dot(jnp.cos(ang), wto_ref[...], preferred_element_type=jnp.float32)
    o_ref[...] = h + b_ref[...]


def _tc1(x_p, rel2, div, W_enc, Wt_e, Wt_o, b01):
    return pl.pallas_call(
        _tc1_body,
        grid=(_NP // _BLK,),
        in_specs=[
            pl.BlockSpec((_BLK, _C), lambda i: (i, 0)),
            pl.BlockSpec((_BLK, 1), lambda i: (i, 0)),
            pl.BlockSpec((1, _C // 2), lambda i: (0, 0)),
            pl.BlockSpec((_C, _C), lambda i: (0, 0)),
            pl.BlockSpec((_C // 2, _C), lambda i: (0, 0)),
            pl.BlockSpec((_C // 2, _C), lambda i: (0, 0)),
            pl.BlockSpec((1, _C), lambda i: (0, 0)),
        ],
        out_specs=pl.BlockSpec((_BLK, _C), lambda i: (i, 0)),
        out_shape=jax.ShapeDtypeStruct((_NP, _C), jnp.float32),
    )(x_p, rel2, div, W_enc, Wt_e, Wt_o, b01)


def _tc3_body(h0_ref, p0_ref, p1_ref, rd_ref, ws_ref, wn_ref, b_ref, uv_ref,
              cv_ref, zu_ref):
    agg = (p0_ref[...] + p1_ref[...]) * rd_ref[...]
    h1 = (jnp.dot(h0_ref[...], ws_ref[...], preferred_element_type=jnp.float32)
          + jnp.dot(agg, wn_ref[...], preferred_element_type=jnp.float32)
          + b_ref[...])
    h1 = jnp.maximum(h1, 0.0)
    zu_ref[...] = jnp.dot(h1, uv_ref[...], preferred_element_type=jnp.float32) + cv_ref[...]


def _tc3(h0, p0, p1, rdeg2, Ws0, Wn0, b0r, uv, cvec):
    return pl.pallas_call(
        _tc3_body,
        grid=(_NP // _BLK,),
        in_specs=[
            pl.BlockSpec((_BLK, _C), lambda i: (i, 0)),
            pl.BlockSpec((_BLK, _C), lambda i: (i, 0)),
            pl.BlockSpec((_BLK, _C), lambda i: (i, 0)),
            pl.BlockSpec((_BLK, 1), lambda i: (i, 0)),
            pl.BlockSpec((_C, _C), lambda i: (0, 0)),
            pl.BlockSpec((_C, _C), lambda i: (0, 0)),
            pl.BlockSpec((1, _C), lambda i: (0, 0)),
            pl.BlockSpec((_C, 2), lambda i: (0, 0)),
            pl.BlockSpec((1, 2), lambda i: (0, 0)),
        ],
        out_specs=pl.BlockSpec((_BLK, 2), lambda i: (i, 0)),
        out_shape=jax.ShapeDtypeStruct((_NP, 2), jnp.float32),
    )(h0, p0, p1, rdeg2, Ws0, Wn0, b0r, uv, cvec)


def kernel(x, edge_index, node_time, seed_time, batch_idx,
           W_enc, b_enc, Wt, bt, Ws0, Wn0, b0, Ws1, Wn1, b1, Wh, bh):
    x_p = jnp.pad(x, ((0, _NP - _N), (0, 0)))
    nt_p = jnp.pad(node_time, (0, _NP - _N))
    bi_p = jnp.pad(batch_idx, (0, _NP - _N))
    src = edge_index[0]
    dst = edge_index[1]

    Wt_e = Wt[0::2]
    Wt_o = Wt[1::2]
    b01 = (b_enc + bt).reshape(1, _C)
    uv = jnp.concatenate([Wn1 @ Wh, Ws1 @ Wh], axis=1)
    cval = b1 @ Wh + bh
    cvec = jnp.concatenate([jnp.zeros((1,), jnp.float32), cval]).reshape(1, 2)
    div = jnp.asarray(
        np.exp(-np.arange(0, _C, 2, dtype=np.float64) * (np.log(10000.0) / _C)),
        jnp.float32).reshape(1, _C // 2)

    rel_t, rdeg = _get_k0()(dst, nt_p, seed_time, bi_p)
    h0 = _tc1(x_p, rel_t.reshape(_NP, 1), div, W_enc, Wt_e, Wt_o, b01)
    parts = _get_sc2()(h0, src, dst)
    zu = _tc3(h0, parts[0], parts[1], rdeg.reshape(_NP, 1),
              Ws0, Wn0, b0.reshape(1, _C), uv, cvec)
    z = zu[:, 0]
    hu = zu[:_S, 1]
    out = _get_sc4()(z, src, dst, hu, rdeg)
    return out.reshape(_S, 1)


# speedup vs baseline: 11.9850x; 1.3572x over previous
import functools

import numpy as np
import jax
import jax.numpy as jnp
from jax import lax
from jax.experimental import pallas as pl
from jax.experimental.pallas import tpu as pltpu
from jax.experimental.pallas import tpu_sc as plsc

_N = 10000
_E = 320000
_C = 128
_S = 1024
_NP = 10240

_NC = 2
_NS = 16
_NW = _NC * _NS

_REL_PER_W = _NP // _NW
_E_PER_SC_TILE = _E // _NS
_E_PER_W = _E // _NW
_B = 80
_NCHUNK = _E_PER_W // _B
_ROWS_PER_TILE = _NP // _NS

def _k0_body(nt_hbm, st_hbm, bi_hbm, rel_out, st_v, bi_v, nt_v, rel_v):
    cid = lax.axis_index("c")
    sid = lax.axis_index("s")
    wid = sid * _NC + cid

    pltpu.sync_copy(st_hbm, st_v)
    pltpu.sync_copy(bi_hbm.at[pl.ds(wid * _REL_PER_W, _REL_PER_W)], bi_v)
    pltpu.sync_copy(nt_hbm.at[pl.ds(wid * _REL_PER_W, _REL_PER_W)], nt_v)

    def rel_step(i, _):
        b16 = bi_v[pl.ds(i * 16, 16)]
        st16 = plsc.load_gather(st_v, [b16])
        nt16 = nt_v[pl.ds(i * 16, 16)]
        rel_v[pl.ds(i * 16, 16)] = (st16 - nt16).astype(jnp.float32) / 86400.0
        return _

    lax.fori_loop(0, _REL_PER_W // 16, rel_step, None)
    pltpu.sync_copy(rel_v, rel_out.at[pl.ds(wid * _REL_PER_W, _REL_PER_W)])


@functools.cache
def _get_k0():
  return functools.partial(
    pl.kernel,
    mesh=plsc.VectorSubcoreMesh(core_axis_name="c", subcore_axis_name="s"),
    compiler_params=pltpu.CompilerParams(needs_layout_passes=False, use_tc_tiling_on_sc=False),
    out_type=jax.ShapeDtypeStruct((_NP,), jnp.float32),
    scratch_types=[
        pltpu.VMEM((_S,), jnp.int32),
        pltpu.VMEM((_REL_PER_W,), jnp.int32),
        pltpu.VMEM((_REL_PER_W,), jnp.int32),
        pltpu.VMEM((_REL_PER_W,), jnp.float32),
    ],
  )(_k0_body)


_NBUF = 5
_CH = _C // _NC
_NCH2 = _E_PER_SC_TILE // _B
_NGRP = _NCH2 // _NBUF


def _sc2_body(h0s_hbm, src3_hbm, dst3_hbm, out_hbm, deg_out,
              src2, dst2, rows0, rows1, rows2, rows3, rows4, zbuf,
              deg_v, acc_sh, sem0, sem1, sem2, sem3, sem4):
    cid = lax.axis_index("c")
    sid = lax.axis_index("s")
    rows = [rows0, rows1, rows2, rows3, rows4]
    sems = [sem0, sem1, sem2, sem3, sem4]
    h0_half = h0s_hbm.at[cid]

    pltpu.sync_copy(src3_hbm.at[sid], src2)
    pltpu.sync_copy(dst3_hbm.at[sid], dst2)
    for b in range(_NBUF):
        pltpu.async_copy(h0_half.at[src2.at[b]], rows[b], sems[b])

    zeros16 = jnp.zeros((16,), jnp.float32)

    def zb_step(r, _):
        for j in range(_CH // 16):
            zbuf[r, pl.ds(j * 16, 16)] = zeros16
        return _

    lax.fori_loop(0, _B, zb_step, None)
    for j in range(_ROWS_PER_TILE // _B):
        pltpu.sync_copy(zbuf, acc_sh.at[pl.ds(sid * _ROWS_PER_TILE + j * _B, _B)])

    def zd_step(i, _):
        deg_v[pl.ds(i * 16, 16)] = zeros16
        return _

    lax.fori_loop(0, _NP // 16, zd_step, None)
    plsc.subcore_barrier()

    ones16 = jnp.full((16,), 1.0, jnp.float32)

    def grp_step(g, _):
        for b in range(_NBUF):
            chunk = g * _NBUF + b
            pltpu.make_async_copy(h0_half.at[src2.at[chunk]], rows[b], sems[b]).wait()
            pltpu.sync_copy(rows[b], acc_sh.at[dst2.at[chunk]], add=True)
            nxt = chunk + _NBUF

            @pl.when(nxt < _NCH2)
            def _refire():
                pltpu.async_copy(h0_half.at[src2.at[nxt]], rows[b], sems[b])

            for j in range(_B // 16):
                d16 = dst2[chunk, pl.ds(j * 16, 16)]
                plsc.addupdate_scatter(deg_v, [d16], ones16)
        return _

    lax.fori_loop(0, _NGRP, grp_step, None)

    @pl.when(cid == 0)
    def _write_deg():
        pltpu.sync_copy(deg_v, deg_out.at[sid])

    plsc.subcore_barrier()

    r0 = sid * _ROWS_PER_TILE

    def wb_step(j, _):
        rr = r0 + j * _B
        pltpu.sync_copy(acc_sh.at[pl.ds(rr, _B)], rows0)
        pltpu.sync_copy(rows0, out_hbm.at[cid, pl.ds(rr, _B)])
        return _

    lax.fori_loop(0, _ROWS_PER_TILE // _B, wb_step, None)


@functools.cache
def _get_sc2():
  return functools.partial(
    pl.kernel,
    mesh=plsc.VectorSubcoreMesh(core_axis_name="c", subcore_axis_name="s"),
    compiler_params=pltpu.CompilerParams(needs_layout_passes=False, use_tc_tiling_on_sc=False),
    out_type=(
        jax.ShapeDtypeStruct((_NC, _NP, _CH), jnp.float32),
        jax.ShapeDtypeStruct((_NS, _NP), jnp.float32),
    ),
    scratch_types=[
        pltpu.VMEM((_NCH2, _B), jnp.int32),
        pltpu.VMEM((_NCH2, _B), jnp.int32),
        pltpu.VMEM((_B, _CH), jnp.float32),
        pltpu.VMEM((_B, _CH), jnp.float32),
        pltpu.VMEM((_B, _CH), jnp.float32),
        pltpu.VMEM((_B, _CH), jnp.float32),
        pltpu.VMEM((_B, _CH), jnp.float32),
        pltpu.VMEM((_B, _CH), jnp.float32),
        pltpu.VMEM((_NP,), jnp.float32),
        pltpu.VMEM_SHARED((_NP, _CH), jnp.float32),
        pltpu.SemaphoreType.DMA,
        pltpu.SemaphoreType.DMA,
        pltpu.SemaphoreType.DMA,
        pltpu.SemaphoreType.DMA,
        pltpu.SemaphoreType.DMA,
    ],
  )(_sc2_body)


def _sc4_body(z_hbm, src_hbm, dst_hbm, hu_hbm, rdeg_hbm, out_hbm,
              z_v, src_v, dst_v, zacc_v, hu_v, rd_v, zslice_v, ztmp_v, o_v,
              zs_sh):
    cid = lax.axis_index("c")
    sid = lax.axis_index("s")

    zeros16 = jnp.zeros((16,), jnp.float32)

    def z_step(i, _):
        zacc_v[pl.ds(i * 16, 16)] = zeros16
        return _

    lax.fori_loop(0, _S // 16, z_step, None)

    pltpu.sync_copy(z_hbm, z_v)
    pltpu.sync_copy(src_hbm.at[pl.ds(sid * _E_PER_SC_TILE, _E_PER_SC_TILE)], src_v)
    pltpu.sync_copy(dst_hbm.at[pl.ds(sid * _E_PER_SC_TILE, _E_PER_SC_TILE)], dst_v)

    def e_step(i, _):
        s16 = src_v[pl.ds(i * 16, 16)]
        d16 = dst_v[pl.ds(i * 16, 16)]
        zz = plsc.load_gather(z_v, [s16])
        m = d16 < _S
        dsafe = jnp.where(m, d16, 0)
        plsc.addupdate_scatter(zacc_v, [dsafe], zz, mask=m)
        return _

    lax.fori_loop(0, _E_PER_SC_TILE // 16, e_step, None)
    pltpu.sync_copy(zacc_v, zs_sh.at[sid])
    plsc.subcore_barrier()

    spw = _S // _NS
    pltpu.sync_copy(zs_sh.at[0, pl.ds(sid * spw, spw)], zslice_v)

    def zc_step(j, _):
        pltpu.sync_copy(zs_sh.at[j, pl.ds(sid * spw, spw)], ztmp_v)

        def za_step(i, __):
            zslice_v[pl.ds(i * 16, 16)] += ztmp_v[pl.ds(i * 16, 16)]
            return __

        lax.fori_loop(0, spw // 16, za_step, None)
        return _

    lax.fori_loop(1, _NS, zc_step, None)
    pltpu.sync_copy(hu_hbm.at[pl.ds(sid * spw, spw)], hu_v)
    pltpu.sync_copy(rdeg_hbm.at[pl.ds(sid * spw, spw)], rd_v)

    def f_step(i, _):
        o_v[pl.ds(i * 16, 16)] = (hu_v[pl.ds(i * 16, 16)]
                                  + zslice_v[pl.ds(i * 16, 16)]
                                  * rd_v[pl.ds(i * 16, 16)])
        return _

    lax.fori_loop(0, spw // 16, f_step, None)

    @pl.when(cid == 0)
    def _write_out():
        pltpu.sync_copy(o_v, out_hbm.at[pl.ds(sid * spw, spw)])


@functools.cache
def _get_sc4():
  return functools.partial(
    pl.kernel,
    mesh=plsc.VectorSubcoreMesh(core_axis_name="c", subcore_axis_name="s"),
    compiler_params=pltpu.CompilerParams(needs_layout_passes=False, use_tc_tiling_on_sc=False),
    out_type=jax.ShapeDtypeStruct((_S,), jnp.float32),
    scratch_types=[
        pltpu.VMEM((_NP,), jnp.float32),
        pltpu.VMEM((_E_PER_SC_TILE,), jnp.int32),
        pltpu.VMEM((_E_PER_SC_TILE,), jnp.int32),
        pltpu.VMEM((_S,), jnp.float32),
        pltpu.VMEM((_S // _NS,), jnp.float32),
        pltpu.VMEM((_S // _NS,), jnp.float32),
        pltpu.VMEM((_S // _NS,), jnp.float32),
        pltpu.VMEM((_S // _NS,), jnp.float32),
        pltpu.VMEM((_S // _NS,), jnp.float32),
        pltpu.VMEM_SHARED((_NS, _S), jnp.float32),
    ],
  )(_sc4_body)


_BLK = 2048


def _tc1_body(x_ref, rel_ref, div_ref, we_ref, wte_ref, wto_ref, b_ref, o_ref):
    ang = rel_ref[...] * div_ref[...]
    h = jnp.dot(x_ref[...], we_ref[...], preferred_element_type=jnp.float32)
    h = h + jnp.dot(jnp.sin(ang), wte_ref[...], preferred_element_type=jnp.float32)
    h = h + jnp.dot(jnp.cos(ang), wto_ref[...], preferred_element_type=jnp.float32)
    h = h + b_ref[...]
    o_ref[0] = h[:, :_CH]
    o_ref[1] = h[:, _CH:]


def _tc1(x_p, rel2, div, W_enc, Wt_e, Wt_o, b01):
    return pl.pallas_call(
        _tc1_body,
        grid=(_NP // _BLK,),
        in_specs=[
            pl.BlockSpec((_BLK, _C), lambda i: (i, 0)),
            pl.BlockSpec((_BLK, 1), lambda i: (i, 0)),
            pl.BlockSpec((1, _C // 2), lambda i: (0, 0)),
            pl.BlockSpec((_C, _C), lambda i: (0, 0)),
            pl.BlockSpec((_C // 2, _C), lambda i: (0, 0)),
            pl.BlockSpec((_C // 2, _C), lambda i: (0, 0)),
            pl.BlockSpec((1, _C), lambda i: (0, 0)),
        ],
        out_specs=pl.BlockSpec((_NC, _BLK, _CH), lambda i: (0, i, 0)),
        out_shape=jax.ShapeDtypeStruct((_NC, _NP, _CH), jnp.float32),
    )(x_p, rel2, div, W_enc, Wt_e, Wt_o, b01)


def _tc3_body(h0s_ref, ps_ref, dg_ref, ws_ref, wn_ref, b_ref,
              uv_ref, cv_ref, zu_ref, rd_ref):
    rd = 1.0 / jnp.maximum(jnp.sum(dg_ref[...], axis=0), 1.0)
    rd_ref[...] = rd
    h0 = jnp.concatenate([h0s_ref[0], h0s_ref[1]], axis=1)
    agg = jnp.concatenate([ps_ref[0], ps_ref[1]], axis=1) * rd
    h1 = (jnp.dot(h0, ws_ref[...], preferred_element_type=jnp.float32)
          + jnp.dot(agg, wn_ref[...], preferred_element_type=jnp.float32)
          + b_ref[...])
    h1 = jnp.maximum(h1, 0.0)
    zu_ref[...] = jnp.dot(h1, uv_ref[...], preferred_element_type=jnp.float32) + cv_ref[...]


def _tc3(h0s, ps, dg, Ws0, Wn0, b0r, uv, cvec):
    return pl.pallas_call(
        _tc3_body,
        grid=(_NP // _BLK,),
        in_specs=[
            pl.BlockSpec((_NC, _BLK, _CH), lambda i: (0, i, 0)),
            pl.BlockSpec((_NC, _BLK, _CH), lambda i: (0, i, 0)),
            pl.BlockSpec((_NS, _BLK, 1), lambda i: (0, i, 0)),
            pl.BlockSpec((_C, _C), lambda i: (0, 0)),
            pl.BlockSpec((_C, _C), lambda i: (0, 0)),
            pl.BlockSpec((1, _C), lambda i: (0, 0)),
            pl.BlockSpec((_C, 2), lambda i: (0, 0)),
            pl.BlockSpec((1, 2), lambda i: (0, 0)),
        ],
        out_specs=[
            pl.BlockSpec((_BLK, 2), lambda i: (i, 0)),
            pl.BlockSpec((_BLK, 1), lambda i: (i, 0)),
        ],
        out_shape=[
            jax.ShapeDtypeStruct((_NP, 2), jnp.float32),
            jax.ShapeDtypeStruct((_NP, 1), jnp.float32),
        ],
    )(h0s, ps, dg, Ws0, Wn0, b0r, uv, cvec)


def kernel(x, edge_index, node_time, seed_time, batch_idx,
           W_enc, b_enc, Wt, bt, Ws0, Wn0, b0, Ws1, Wn1, b1, Wh, bh):
    x_p = jnp.pad(x, ((0, _NP - _N), (0, 0)))
    nt_p = jnp.pad(node_time, (0, _NP - _N))
    bi_p = jnp.pad(batch_idx, (0, _NP - _N))
    src = edge_index[0]
    dst = edge_index[1]

    Wt_e = Wt[0::2]
    Wt_o = Wt[1::2]
    b01 = (b_enc + bt).reshape(1, _C)
    uv = jnp.concatenate([Wn1 @ Wh, Ws1 @ Wh], axis=1)
    cval = b1 @ Wh + bh
    cvec = jnp.concatenate([jnp.zeros((1,), jnp.float32), cval]).reshape(1, 2)
    div = jnp.asarray(
        np.exp(-np.arange(0, _C, 2, dtype=np.float64) * (np.log(10000.0) / _C)),
        jnp.float32).reshape(1, _C // 2)

    rel_t = _get_k0()(nt_p, seed_time, bi_p)
    h0s = _tc1(x_p, rel_t.reshape(_NP, 1), div, W_enc, Wt_e, Wt_o, b01)
    src3 = src.reshape(_NS, _NCH2, _B)
    dst3 = dst.reshape(_NS, _NCH2, _B)
    parts, degp = _get_sc2()(h0s, src3, dst3)
    zu, rd = _tc3(h0s, parts, degp.reshape(_NS, _NP, 1),
                  Ws0, Wn0, b0.reshape(1, _C), uv, cvec)
    z = zu[:, 0]
    hu = zu[:_S, 1]
    rd1k = rd[:_S, 0]
    out = _get_sc4()(z, src, dst, hu, rd1k)
    return out.reshape(_S, 1)


# speedup vs baseline: 17.6131x; 1.4696x over previous
import functools

import numpy as np
import jax
import jax.numpy as jnp
from jax import lax
from jax.experimental import pallas as pl
from jax.experimental.pallas import tpu as pltpu
from jax.experimental.pallas import tpu_sc as plsc

_N = 10000
_E = 320000
_C = 128
_S = 1024
_NP = 10240

_NC = 2
_NS = 16
_NW = _NC * _NS

_REL_PER_W = _NP // _NW
_E_PER_SC_TILE = _E // _NS
_E_PER_W = _E // _NW
_B = 80
_NCHUNK = _E_PER_W // _B
_ROWS_PER_TILE = _NP // _NS

def _k0_body(nt_hbm, st_hbm, bi_hbm, rel_out, st_v, bi_v, nt_v, rel_v):
    cid = lax.axis_index("c")
    sid = lax.axis_index("s")
    wid = sid * _NC + cid

    pltpu.sync_copy(st_hbm, st_v)
    pltpu.sync_copy(bi_hbm.at[pl.ds(wid * _REL_PER_W, _REL_PER_W)], bi_v)
    pltpu.sync_copy(nt_hbm.at[pl.ds(wid * _REL_PER_W, _REL_PER_W)], nt_v)

    def rel_step(i, _):
        b16 = bi_v[pl.ds(i * 16, 16)]
        st16 = plsc.load_gather(st_v, [b16])
        nt16 = nt_v[pl.ds(i * 16, 16)]
        rel_v[pl.ds(i * 16, 16)] = (st16 - nt16).astype(jnp.float32) / 86400.0
        return _

    lax.fori_loop(0, _REL_PER_W // 16, rel_step, None)
    pltpu.sync_copy(rel_v, rel_out.at[pl.ds(wid * _REL_PER_W, _REL_PER_W)])


@functools.cache
def _get_k0():
  return functools.partial(
    pl.kernel,
    mesh=plsc.VectorSubcoreMesh(core_axis_name="c", subcore_axis_name="s"),
    compiler_params=pltpu.CompilerParams(needs_layout_passes=False, use_tc_tiling_on_sc=False),
    out_type=jax.ShapeDtypeStruct((_NP,), jnp.float32),
    scratch_types=[
        pltpu.VMEM((_S,), jnp.int32),
        pltpu.VMEM((_REL_PER_W,), jnp.int32),
        pltpu.VMEM((_REL_PER_W,), jnp.int32),
        pltpu.VMEM((_REL_PER_W,), jnp.float32),
    ],
  )(_k0_body)


_NBUF = 5
_CH = _C // _NC
_NCH2 = _E_PER_SC_TILE // _B
_NGRP = _NCH2 // _NBUF


def _sc2_body(h0s_hbm, src3_hbm, dst3_hbm, out_hbm, deg_out,
              src2, dst2, rows0, rows1, rows2, rows3, rows4, zbuf,
              deg_v, acc_sh, sem0, sem1, sem2, sem3, sem4):
    cid = lax.axis_index("c")
    sid = lax.axis_index("s")
    rows = [rows0, rows1, rows2, rows3, rows4]
    sems = [sem0, sem1, sem2, sem3, sem4]
    h0_half = h0s_hbm.at[cid]

    pltpu.sync_copy(src3_hbm.at[sid], src2)
    pltpu.sync_copy(dst3_hbm.at[sid], dst2)
    for b in range(_NBUF):
        pltpu.async_copy(h0_half.at[src2.at[b]], rows[b], sems[b])

    zeros16 = jnp.zeros((16,), jnp.float32)

    def zb_step(r, _):
        for j in range(_CH // 16):
            zbuf[r, pl.ds(j * 16, 16)] = zeros16
        return _

    lax.fori_loop(0, _B, zb_step, None)
    for j in range(_ROWS_PER_TILE // _B):
        pltpu.sync_copy(zbuf, acc_sh.at[pl.ds(sid * _ROWS_PER_TILE + j * _B, _B)])

    def zd_step(i, _):
        deg_v[pl.ds(i * 16, 16)] = zeros16
        return _

    lax.fori_loop(0, _NP // 16, zd_step, None)
    plsc.subcore_barrier()

    ones16 = jnp.full((16,), 1.0, jnp.float32)

    def grp_step(g, _):
        for b in range(_NBUF):
            chunk = g * _NBUF + b
            pltpu.make_async_copy(h0_half.at[src2.at[chunk]], rows[b], sems[b]).wait()
            pltpu.sync_copy(rows[b], acc_sh.at[dst2.at[chunk]], add=True)
            nxt = chunk + _NBUF

            @pl.when(nxt < _NCH2)
            def _refire():
                pltpu.async_copy(h0_half.at[src2.at[nxt]], rows[b], sems[b])

            for j in range(_B // 16):
                d16 = dst2[chunk, pl.ds(j * 16, 16)]
                plsc.addupdate_scatter(deg_v, [d16], ones16)
        return _

    lax.fori_loop(0, _NGRP, grp_step, None)

    @pl.when(cid == 0)
    def _write_deg():
        pltpu.sync_copy(deg_v, deg_out.at[sid])

    plsc.subcore_barrier()

    r0 = sid * _ROWS_PER_TILE

    def wb_step(j, _):
        rr = r0 + j * _B
        pltpu.sync_copy(acc_sh.at[pl.ds(rr, _B)], rows0)
        pltpu.sync_copy(rows0, out_hbm.at[cid, pl.ds(rr, _B)])
        return _

    lax.fori_loop(0, _ROWS_PER_TILE // _B, wb_step, None)


@functools.cache
def _get_sc2():
  return functools.partial(
    pl.kernel,
    mesh=plsc.VectorSubcoreMesh(core_axis_name="c", subcore_axis_name="s"),
    compiler_params=pltpu.CompilerParams(needs_layout_passes=False, use_tc_tiling_on_sc=False),
    out_type=(
        jax.ShapeDtypeStruct((_NC, _NP, _CH), jnp.float32),
        jax.ShapeDtypeStruct((_NS, _NP), jnp.float32),
    ),
    scratch_types=[
        pltpu.VMEM((_NCH2, _B), jnp.int32),
        pltpu.VMEM((_NCH2, _B), jnp.int32),
        pltpu.VMEM((_B, _CH), jnp.float32),
        pltpu.VMEM((_B, _CH), jnp.float32),
        pltpu.VMEM((_B, _CH), jnp.float32),
        pltpu.VMEM((_B, _CH), jnp.float32),
        pltpu.VMEM((_B, _CH), jnp.float32),
        pltpu.VMEM((_B, _CH), jnp.float32),
        pltpu.VMEM((_NP,), jnp.float32),
        pltpu.VMEM_SHARED((_NP, _CH), jnp.float32),
        pltpu.SemaphoreType.DMA,
        pltpu.SemaphoreType.DMA,
        pltpu.SemaphoreType.DMA,
        pltpu.SemaphoreType.DMA,
        pltpu.SemaphoreType.DMA,
    ],
  )(_sc2_body)


def _sc4_body(z_hbm, src_hbm, dst_hbm, hu_hbm, rdeg_hbm, out_hbm,
              z_v, src_v, dst_v, zacc_v, hu_v, rd_v, zslice_v, ztmp_v, o_v,
              zs_sh):
    cid = lax.axis_index("c")
    sid = lax.axis_index("s")

    zeros16 = jnp.zeros((16,), jnp.float32)

    def z_step(i, _):
        zacc_v[pl.ds(i * 16, 16)] = zeros16
        return _

    lax.fori_loop(0, _S // 16, z_step, None)

    pltpu.sync_copy(z_hbm, z_v)
    pltpu.sync_copy(src_hbm.at[pl.ds(sid * _E_PER_SC_TILE, _E_PER_SC_TILE)], src_v)
    pltpu.sync_copy(dst_hbm.at[pl.ds(sid * _E_PER_SC_TILE, _E_PER_SC_TILE)], dst_v)

    def e_step(i, _):
        s16 = src_v[pl.ds(i * 16, 16)]
        d16 = dst_v[pl.ds(i * 16, 16)]
        zz = plsc.load_gather(z_v, [s16])
        m = d16 < _S
        dsafe = jnp.where(m, d16, 0)
        plsc.addupdate_scatter(zacc_v, [dsafe], zz, mask=m)
        return _

    lax.fori_loop(0, _E_PER_SC_TILE // 16, e_step, None)
    pltpu.sync_copy(zacc_v, zs_sh.at[sid])
    plsc.subcore_barrier()

    spw = _S // _NS
    pltpu.sync_copy(zs_sh.at[0, pl.ds(sid * spw, spw)], zslice_v)

    def zc_step(j, _):
        pltpu.sync_copy(zs_sh.at[j, pl.ds(sid * spw, spw)], ztmp_v)

        def za_step(i, __):
            zslice_v[pl.ds(i * 16, 16)] += ztmp_v[pl.ds(i * 16, 16)]
            return __

        lax.fori_loop(0, spw // 16, za_step, None)
        return _

    lax.fori_loop(1, _NS, zc_step, None)
    pltpu.sync_copy(hu_hbm.at[pl.ds(sid * spw, spw)], hu_v)
    pltpu.sync_copy(rdeg_hbm.at[pl.ds(sid * spw, spw)], rd_v)

    def f_step(i, _):
        o_v[pl.ds(i * 16, 16)] = (hu_v[pl.ds(i * 16, 16)]
                                  + zslice_v[pl.ds(i * 16, 16)]
                                  * rd_v[pl.ds(i * 16, 16)])
        return _

    lax.fori_loop(0, spw // 16, f_step, None)

    @pl.when(cid == 0)
    def _write_out():
        pltpu.sync_copy(o_v, out_hbm.at[pl.ds(sid * spw, spw)])


@functools.cache
def _get_sc4():
  return functools.partial(
    pl.kernel,
    mesh=plsc.VectorSubcoreMesh(core_axis_name="c", subcore_axis_name="s"),
    compiler_params=pltpu.CompilerParams(needs_layout_passes=False, use_tc_tiling_on_sc=False),
    out_type=jax.ShapeDtypeStruct((_S,), jnp.float32),
    scratch_types=[
        pltpu.VMEM((_NP,), jnp.float32),
        pltpu.VMEM((_E_PER_SC_TILE,), jnp.int32),
        pltpu.VMEM((_E_PER_SC_TILE,), jnp.int32),
        pltpu.VMEM((_S,), jnp.float32),
        pltpu.VMEM((_S // _NS,), jnp.float32),
        pltpu.VMEM((_S // _NS,), jnp.float32),
        pltpu.VMEM((_S // _NS,), jnp.float32),
        pltpu.VMEM((_S // _NS,), jnp.float32),
        pltpu.VMEM((_S // _NS,), jnp.float32),
        pltpu.VMEM_SHARED((_NS, _S), jnp.float32),
    ],
  )(_sc4_body)


_BLK = 2048


def _tc1_body(x_ref, rel_ref, div_ref, we_ref, wte_ref, wto_ref, b_ref, o_ref):
    ang = rel_ref[...] * div_ref[...]
    h = jnp.dot(x_ref[...], we_ref[...], preferred_element_type=jnp.float32)
    h = h + jnp.dot(jnp.sin(ang), wte_ref[...], preferred_element_type=jnp.float32)
    h = h + jnp.dot(jnp.cos(ang), wto_ref[...], preferred_element_type=jnp.float32)
    h = h + b_ref[...]
    o_ref[0] = h[:, :_CH]
    o_ref[1] = h[:, _CH:]


def _tc1(x_in, rel2, div, W_enc, Wt_e, Wt_o, b01):
    return pl.pallas_call(
        _tc1_body,
        grid=(_NP // _BLK,),
        in_specs=[
            pl.BlockSpec((_BLK, _C), lambda i: (i, 0)),
            pl.BlockSpec((_BLK, 1), lambda i: (i, 0)),
            pl.BlockSpec((1, _C // 2), lambda i: (0, 0)),
            pl.BlockSpec((_C, _C), lambda i: (0, 0)),
            pl.BlockSpec((_C // 2, _C), lambda i: (0, 0)),
            pl.BlockSpec((_C // 2, _C), lambda i: (0, 0)),
            pl.BlockSpec((1, _C), lambda i: (0, 0)),
        ],
        out_specs=pl.BlockSpec((_NC, _BLK, _CH), lambda i: (0, i, 0)),
        out_shape=jax.ShapeDtypeStruct((_NC, _NP, _CH), jnp.float32),
    )(x_in, rel2, div, W_enc, Wt_e, Wt_o, b01)


def _tc3_body(h0s_ref, ps_ref, dg_ref, ws_ref, wn_ref, b_ref,
              ws1_ref, wn1_ref, wh_ref, b1_ref, bh_ref, zu_ref, rd_ref):
    ones_c = jnp.ones((_NS, 1), jnp.float32)
    deg_col = lax.dot_general(dg_ref[...], ones_c, (((0,), (0,)), ((), ())),
                              preferred_element_type=jnp.float32)
    rd = 1.0 / jnp.maximum(deg_col, 1.0)
    rd_ref[...] = rd
    h0 = jnp.concatenate([h0s_ref[0], h0s_ref[1]], axis=1)
    aggs = jnp.concatenate([ps_ref[0], ps_ref[1]], axis=1)
    h1 = (jnp.dot(h0, ws_ref[...], preferred_element_type=jnp.float32)
          + jnp.dot(aggs, wn_ref[...], preferred_element_type=jnp.float32) * rd
          + b_ref[...])
    h1 = jnp.maximum(h1, 0.0)
    wh = wh_ref[...]
    uv = jnp.concatenate(
        [jnp.dot(wn1_ref[...], wh, preferred_element_type=jnp.float32),
         jnp.dot(ws1_ref[...], wh, preferred_element_type=jnp.float32)], axis=1)
    cval = jnp.dot(b1_ref[...], wh, preferred_element_type=jnp.float32) + bh_ref[...]
    cvec = jnp.concatenate([jnp.zeros((1, 1), jnp.float32), cval], axis=1)
    zu_ref[...] = jnp.dot(h1, uv, preferred_element_type=jnp.float32) + cvec


def _tc3(h0s, ps, dg, Ws0, Wn0, b0r, Ws1, Wn1, Wh, b1r, bhr):
    return pl.pallas_call(
        _tc3_body,
        grid=(_NP // _BLK,),
        in_specs=[
            pl.BlockSpec((_NC, _BLK, _CH), lambda i: (0, i, 0)),
            pl.BlockSpec((_NC, _BLK, _CH), lambda i: (0, i, 0)),
            pl.BlockSpec((_NS, _BLK), lambda i: (0, i)),
            pl.BlockSpec((_C, _C), lambda i: (0, 0)),
            pl.BlockSpec((_C, _C), lambda i: (0, 0)),
            pl.BlockSpec((1, _C), lambda i: (0, 0)),
            pl.BlockSpec((_C, _C), lambda i: (0, 0)),
            pl.BlockSpec((_C, _C), lambda i: (0, 0)),
            pl.BlockSpec((_C, 1), lambda i: (0, 0)),
            pl.BlockSpec((1, _C), lambda i: (0, 0)),
            pl.BlockSpec((1, 1), lambda i: (0, 0)),
        ],
        out_specs=[
            pl.BlockSpec((_BLK, 2), lambda i: (i, 0)),
            pl.BlockSpec((_BLK, 1), lambda i: (i, 0)),
        ],
        out_shape=[
            jax.ShapeDtypeStruct((_NP, 2), jnp.float32),
            jax.ShapeDtypeStruct((_NP, 1), jnp.float32),
        ],
    )(h0s, ps, dg, Ws0, Wn0, b0r, Ws1, Wn1, Wh, b1r, bhr)


def kernel(x, edge_index, node_time, seed_time, batch_idx,
           W_enc, b_enc, Wt, bt, Ws0, Wn0, b0, Ws1, Wn1, b1, Wh, bh):
    nt_p = jnp.pad(node_time, (0, _NP - _N))
    bi_p = jnp.pad(batch_idx, (0, _NP - _N))
    src = edge_index[0]
    dst = edge_index[1]

    Wt_e = Wt[0::2]
    Wt_o = Wt[1::2]
    b01 = (b_enc + bt).reshape(1, _C)
    div = jnp.asarray(
        np.exp(-np.arange(0, _C, 2, dtype=np.float64) * (np.log(10000.0) / _C)),
        jnp.float32).reshape(1, _C // 2)

    rel_t = _get_k0()(nt_p, seed_time, bi_p)
    h0s = _tc1(x, rel_t.reshape(_NP, 1), div, W_enc, Wt_e, Wt_o, b01)
    src3 = src.reshape(_NS, _NCH2, _B)
    dst3 = dst.reshape(_NS, _NCH2, _B)
    parts, degp = _get_sc2()(h0s, src3, dst3)
    zu, rd = _tc3(h0s, parts, degp, Ws0, Wn0, b0.reshape(1, _C),
                  Ws1, Wn1, Wh, b1.reshape(1, _C), bh.reshape(1, 1))
    z = zu[:, 0]
    hu = zu[:_S, 1]
    rd1k = rd[:_S, 0]
    out = _get_sc4()(z, src, dst, hu, rd1k)
    return out.reshape(_S, 1)


# speedup vs baseline: 18.6240x; 1.0574x over previous
import functools

import numpy as np
import jax
import jax.numpy as jnp
from jax import lax
from jax.experimental import pallas as pl
from jax.experimental.pallas import tpu as pltpu
from jax.experimental.pallas import tpu_sc as plsc

_N = 10000
_E = 320000
_C = 128
_S = 1024
_NP = 10240

_NC = 2
_NS = 16
_NW = _NC * _NS

_REL_PER_W = _NP // _NW
_E_PER_SC_TILE = _E // _NS
_E_PER_W = _E // _NW
_B = 80
_NCHUNK = _E_PER_W // _B
_ROWS_PER_TILE = _NP // _NS

def _k0_body(nt_hbm, st_hbm, bi_hbm, rel_out, st_v, bi_v, nt_v, rel_v):
    cid = lax.axis_index("c")
    sid = lax.axis_index("s")
    wid = sid * _NC + cid

    pltpu.sync_copy(st_hbm, st_v)
    pltpu.sync_copy(bi_hbm.at[pl.ds(wid * _REL_PER_W, _REL_PER_W)], bi_v)
    pltpu.sync_copy(nt_hbm.at[pl.ds(wid * _REL_PER_W, _REL_PER_W)], nt_v)

    def rel_step(i, _):
        b16 = bi_v[pl.ds(i * 16, 16)]
        st16 = plsc.load_gather(st_v, [b16])
        nt16 = nt_v[pl.ds(i * 16, 16)]
        rel_v[pl.ds(i * 16, 16)] = (st16 - nt16).astype(jnp.float32) / 86400.0
        return _

    lax.fori_loop(0, _REL_PER_W // 16, rel_step, None)
    pltpu.sync_copy(rel_v, rel_out.at[pl.ds(wid * _REL_PER_W, _REL_PER_W)])


@functools.cache
def _get_k0():
  return functools.partial(
    pl.kernel,
    mesh=plsc.VectorSubcoreMesh(core_axis_name="c", subcore_axis_name="s"),
    compiler_params=pltpu.CompilerParams(needs_layout_passes=False, use_tc_tiling_on_sc=False),
    out_type=jax.ShapeDtypeStruct((_NP,), jnp.float32),
    scratch_types=[
        pltpu.VMEM((_S,), jnp.int32),
        pltpu.VMEM((_REL_PER_W,), jnp.int32),
        pltpu.VMEM((_REL_PER_W,), jnp.int32),
        pltpu.VMEM((_REL_PER_W,), jnp.float32),
    ],
  )(_k0_body)


_NBUF = 5
_CH = _C // _NC
_NCH2 = _E_PER_SC_TILE // _B
_NGRP = _NCH2 // _NBUF


def _sc2_body(h0s_hbm, e4_hbm, out_hbm, deg_out,
              src2, dst2, rows0, rows1, rows2, rows3, rows4, zbuf,
              deg_v, acc_sh, sem0, sem1, sem2, sem3, sem4):
    cid = lax.axis_index("c")
    sid = lax.axis_index("s")
    rows = [rows0, rows1, rows2, rows3, rows4]
    sems = [sem0, sem1, sem2, sem3, sem4]
    h0_half = h0s_hbm.at[cid]

    pltpu.sync_copy(e4_hbm.at[0, sid], src2)
    pltpu.sync_copy(e4_hbm.at[1, sid], dst2)
    for b in range(_NBUF):
        pltpu.async_copy(h0_half.at[src2.at[b]], rows[b], sems[b])

    zeros16 = jnp.zeros((16,), jnp.float32)

    def zb_step(r, _):
        for j in range(_CH // 16):
            zbuf[r, pl.ds(j * 16, 16)] = zeros16
        return _

    lax.fori_loop(0, _B, zb_step, None)
    for j in range(_ROWS_PER_TILE // _B):
        pltpu.sync_copy(zbuf, acc_sh.at[pl.ds(sid * _ROWS_PER_TILE + j * _B, _B)])

    def zd_step(i, _):
        deg_v[pl.ds(i * 16, 16)] = zeros16
        return _

    lax.fori_loop(0, _NP // 16, zd_step, None)
    plsc.subcore_barrier()

    ones16 = jnp.full((16,), 1.0, jnp.float32)

    def grp_step(g, _):
        for b in range(_NBUF):
            chunk = g * _NBUF + b
            pltpu.make_async_copy(h0_half.at[src2.at[chunk]], rows[b], sems[b]).wait()
            pltpu.sync_copy(rows[b], acc_sh.at[dst2.at[chunk]], add=True)
            nxt = chunk + _NBUF

            @pl.when(nxt < _NCH2)
            def _refire():
                pltpu.async_copy(h0_half.at[src2.at[nxt]], rows[b], sems[b])

            for j in range(_B // 16):
                d16 = dst2[chunk, pl.ds(j * 16, 16)]
                plsc.addupdate_scatter(deg_v, [d16], ones16)
        return _

    lax.fori_loop(0, _NGRP, grp_step, None)

    @pl.when(cid == 0)
    def _write_deg():
        pltpu.sync_copy(deg_v, deg_out.at[sid])

    plsc.subcore_barrier()

    r0 = sid * _ROWS_PER_TILE

    def wb_step(j, _):
        rr = r0 + j * _B
        pltpu.sync_copy(acc_sh.at[pl.ds(rr, _B)], rows0)
        pltpu.sync_copy(rows0, out_hbm.at[cid, pl.ds(rr, _B)])
        return _

    lax.fori_loop(0, _ROWS_PER_TILE // _B, wb_step, None)


@functools.cache
def _get_sc2():
  return functools.partial(
    pl.kernel,
    mesh=plsc.VectorSubcoreMesh(core_axis_name="c", subcore_axis_name="s"),
    compiler_params=pltpu.CompilerParams(needs_layout_passes=False, use_tc_tiling_on_sc=False),
    out_type=(
        jax.ShapeDtypeStruct((_NC, _NP, _CH), jnp.float32),
        jax.ShapeDtypeStruct((_NS, _NP), jnp.float32),
    ),
    scratch_types=[
        pltpu.VMEM((_NCH2, _B), jnp.int32),
        pltpu.VMEM((_NCH2, _B), jnp.int32),
        pltpu.VMEM((_B, _CH), jnp.float32),
        pltpu.VMEM((_B, _CH), jnp.float32),
        pltpu.VMEM((_B, _CH), jnp.float32),
        pltpu.VMEM((_B, _CH), jnp.float32),
        pltpu.VMEM((_B, _CH), jnp.float32),
        pltpu.VMEM((_B, _CH), jnp.float32),
        pltpu.VMEM((_NP,), jnp.float32),
        pltpu.VMEM_SHARED((_NP, _CH), jnp.float32),
        pltpu.SemaphoreType.DMA,
        pltpu.SemaphoreType.DMA,
        pltpu.SemaphoreType.DMA,
        pltpu.SemaphoreType.DMA,
        pltpu.SemaphoreType.DMA,
    ],
  )(_sc2_body)


def _sc4_body(z_hbm, e4_hbm, hu_hbm, rdeg_hbm, out_hbm,
              z_v, src_v, dst_v, zacc_v, hu_v, rd_v, zslice_v, ztmp_v, o_v,
              zs_sh):
    cid = lax.axis_index("c")
    sid = lax.axis_index("s")

    zeros16 = jnp.zeros((16,), jnp.float32)

    def z_step(i, _):
        zacc_v[pl.ds(i * 16, 16)] = zeros16
        return _

    lax.fori_loop(0, _S // 16, z_step, None)

    pltpu.sync_copy(z_hbm, z_v)
    pltpu.sync_copy(e4_hbm.at[0, sid], src_v)
    pltpu.sync_copy(e4_hbm.at[1, sid], dst_v)

    def e_step(r, _):
        for j in range(_B // 16):
            s16 = src_v[r, pl.ds(j * 16, 16)]
            d16 = dst_v[r, pl.ds(j * 16, 16)]
            zz = plsc.load_gather(z_v, [s16])
            m = d16 < _S
            dsafe = jnp.where(m, d16, 0)
            plsc.addupdate_scatter(zacc_v, [dsafe], zz, mask=m)
        return _

    lax.fori_loop(0, _NCH2, e_step, None)
    pltpu.sync_copy(zacc_v, zs_sh.at[sid])
    plsc.subcore_barrier()

    spw = _S // _NS
    pltpu.sync_copy(zs_sh.at[0, pl.ds(sid * spw, spw)], zslice_v)

    def zc_step(j, _):
        pltpu.sync_copy(zs_sh.at[j, pl.ds(sid * spw, spw)], ztmp_v)

        def za_step(i, __):
            zslice_v[pl.ds(i * 16, 16)] += ztmp_v[pl.ds(i * 16, 16)]
            return __

        lax.fori_loop(0, spw // 16, za_step, None)
        return _

    lax.fori_loop(1, _NS, zc_step, None)
    pltpu.sync_copy(hu_hbm.at[pl.ds(sid * spw, spw)], hu_v)
    pltpu.sync_copy(rdeg_hbm.at[pl.ds(sid * spw, spw)], rd_v)

    def f_step(i, _):
        o_v[pl.ds(i * 16, 16)] = (hu_v[pl.ds(i * 16, 16)]
                                  + zslice_v[pl.ds(i * 16, 16)]
                                  * rd_v[pl.ds(i * 16, 16)])
        return _

    lax.fori_loop(0, spw // 16, f_step, None)

    @pl.when(cid == 0)
    def _write_out():
        pltpu.sync_copy(o_v, out_hbm.at[pl.ds(sid * spw, spw)])


@functools.cache
def _get_sc4():
  return functools.partial(
    pl.kernel,
    mesh=plsc.VectorSubcoreMesh(core_axis_name="c", subcore_axis_name="s"),
    compiler_params=pltpu.CompilerParams(needs_layout_passes=False, use_tc_tiling_on_sc=False),
    out_type=jax.ShapeDtypeStruct((_S,), jnp.float32),
    scratch_types=[
        pltpu.VMEM((_NP,), jnp.float32),
        pltpu.VMEM((_NCH2, _B), jnp.int32),
        pltpu.VMEM((_NCH2, _B), jnp.int32),
        pltpu.VMEM((_S,), jnp.float32),
        pltpu.VMEM((_S // _NS,), jnp.float32),
        pltpu.VMEM((_S // _NS,), jnp.float32),
        pltpu.VMEM((_S // _NS,), jnp.float32),
        pltpu.VMEM((_S // _NS,), jnp.float32),
        pltpu.VMEM((_S // _NS,), jnp.float32),
        pltpu.VMEM_SHARED((_NS, _S), jnp.float32),
    ],
  )(_sc4_body)


_BLK = 2048


def _tc1_body(x_ref, rel_ref, div_ref, we_ref, wte_ref, wto_ref, b_ref, o_ref):
    ang = rel_ref[...] * div_ref[...]
    h = jnp.dot(x_ref[...], we_ref[...], preferred_element_type=jnp.float32)
    h = h + jnp.dot(jnp.sin(ang), wte_ref[...], preferred_element_type=jnp.float32)
    h = h + jnp.dot(jnp.cos(ang), wto_ref[...], preferred_element_type=jnp.float32)
    h = h + b_ref[...]
    o_ref[0] = h[:, :_CH]
    o_ref[1] = h[:, _CH:]


def _tc1(x_in, rel2, div, W_enc, Wt_e, Wt_o, b01):
    return pl.pallas_call(
        _tc1_body,
        grid=(_NP // _BLK,),
        in_specs=[
            pl.BlockSpec((_BLK, _C), lambda i: (i, 0)),
            pl.BlockSpec((_BLK, 1), lambda i: (i, 0)),
            pl.BlockSpec((1, _C // 2), lambda i: (0, 0)),
            pl.BlockSpec((_C, _C), lambda i: (0, 0)),
            pl.BlockSpec((_C // 2, _C), lambda i: (0, 0)),
            pl.BlockSpec((_C // 2, _C), lambda i: (0, 0)),
            pl.BlockSpec((1, _C), lambda i: (0, 0)),
        ],
        out_specs=pl.BlockSpec((_NC, _BLK, _CH), lambda i: (0, i, 0)),
        out_shape=jax.ShapeDtypeStruct((_NC, _NP, _CH), jnp.float32),
    )(x_in, rel2, div, W_enc, Wt_e, Wt_o, b01)


def _tc3_body(h0s_ref, ps_ref, dg_ref, ws_ref, wn_ref, b_ref,
              ws1_ref, wn1_ref, wh_ref, b1_ref, bh_ref, zu_ref, rd_ref):
    ones_c = jnp.ones((_NS, 1), jnp.float32)
    deg_col = lax.dot_general(dg_ref[...], ones_c, (((0,), (0,)), ((), ())),
                              preferred_element_type=jnp.float32)
    rd = 1.0 / jnp.maximum(deg_col, 1.0)
    rd_ref[...] = rd
    h0 = jnp.concatenate([h0s_ref[0], h0s_ref[1]], axis=1)
    aggs = jnp.concatenate([ps_ref[0], ps_ref[1]], axis=1)
    h1 = (jnp.dot(h0, ws_ref[...], preferred_element_type=jnp.float32)
          + jnp.dot(aggs, wn_ref[...], preferred_element_type=jnp.float32) * rd
          + b_ref[...])
    h1 = jnp.maximum(h1, 0.0)
    wh = wh_ref[...]
    uv = jnp.concatenate(
        [jnp.dot(wn1_ref[...], wh, preferred_element_type=jnp.float32),
         jnp.dot(ws1_ref[...], wh, preferred_element_type=jnp.float32)], axis=1)
    cval = jnp.dot(b1_ref[...], wh, preferred_element_type=jnp.float32) + bh_ref[...]
    cvec = jnp.concatenate([jnp.zeros((1, 1), jnp.float32), cval], axis=1)
    zu_ref[...] = jnp.dot(h1, uv, preferred_element_type=jnp.float32) + cvec


def _tc3(h0s, ps, dg, Ws0, Wn0, b0r, Ws1, Wn1, Wh, b1r, bhr):
    return pl.pallas_call(
        _tc3_body,
        grid=(_NP // _BLK,),
        in_specs=[
            pl.BlockSpec((_NC, _BLK, _CH), lambda i: (0, i, 0)),
            pl.BlockSpec((_NC, _BLK, _CH), lambda i: (0, i, 0)),
            pl.BlockSpec((_NS, _BLK), lambda i: (0, i)),
            pl.BlockSpec((_C, _C), lambda i: (0, 0)),
            pl.BlockSpec((_C, _C), lambda i: (0, 0)),
            pl.BlockSpec((1, _C), lambda i: (0, 0)),
            pl.BlockSpec((_C, _C), lambda i: (0, 0)),
            pl.BlockSpec((_C, _C), lambda i: (0, 0)),
            pl.BlockSpec((_C, 1), lambda i: (0, 0)),
            pl.BlockSpec((1, _C), lambda i: (0, 0)),
            pl.BlockSpec((1, 1), lambda i: (0, 0)),
        ],
        out_specs=[
            pl.BlockSpec((_BLK, 2), lambda i: (i, 0)),
            pl.BlockSpec((_BLK, 1), lambda i: (i, 0)),
        ],
        out_shape=[
            jax.ShapeDtypeStruct((_NP, 2), jnp.float32),
            jax.ShapeDtypeStruct((_NP, 1), jnp.float32),
        ],
    )(h0s, ps, dg, Ws0, Wn0, b0r, Ws1, Wn1, Wh, b1r, bhr)


def kernel(x, edge_index, node_time, seed_time, batch_idx,
           W_enc, b_enc, Wt, bt, Ws0, Wn0, b0, Ws1, Wn1, b1, Wh, bh):
    nt_p = jnp.pad(node_time, (0, _NP - _N))
    bi_p = jnp.pad(batch_idx, (0, _NP - _N))
    src = edge_index[0]
    dst = edge_index[1]

    Wt_e = Wt[0::2]
    Wt_o = Wt[1::2]
    b01 = (b_enc + bt).reshape(1, _C)
    div = jnp.asarray(
        np.exp(-np.arange(0, _C, 2, dtype=np.float64) * (np.log(10000.0) / _C)),
        jnp.float32).reshape(1, _C // 2)

    rel_t = _get_k0()(nt_p, seed_time, bi_p)
    h0s = _tc1(x, rel_t.reshape(_NP, 1), div, W_enc, Wt_e, Wt_o, b01)
    e4 = edge_index.reshape(2, _NS, _NCH2, _B)
    parts, degp = _get_sc2()(h0s, e4)
    zu, rd = _tc3(h0s, parts, degp, Ws0, Wn0, b0.reshape(1, _C),
                  Ws1, Wn1, Wh, b1.reshape(1, _C), bh.reshape(1, 1))
    z = zu[:, 0]
    hu = zu[:_S, 1]
    rd1k = rd[:_S, 0]
    out = _get_sc4()(z, e4, hu, rd1k)
    return out.reshape(_S, 1)


# speedup vs baseline: 20.7202x; 1.1126x over previous
import functools

import numpy as np
import jax
import jax.numpy as jnp
from jax import lax
from jax.experimental import pallas as pl
from jax.experimental.pallas import tpu as pltpu
from jax.experimental.pallas import tpu_sc as plsc

_N = 10000
_E = 320000
_C = 128
_S = 1024
_NP = 10240

_NC = 2
_NS = 16
_NW = _NC * _NS

_REL_PER_W = _NP // _NW
_E_PER_SC_TILE = _E // _NS
_E_PER_W = _E // _NW
_B = 80
_NCHUNK = _E_PER_W // _B
_ROWS_PER_TILE = _NP // _NS

def _k0_body(nt_hbm, st_hbm, bi_hbm, rel_out, st_v, bi_v, nt_v, rel_v):
    cid = lax.axis_index("c")
    sid = lax.axis_index("s")
    wid = sid * _NC + cid

    pltpu.sync_copy(st_hbm, st_v)
    pltpu.sync_copy(bi_hbm.at[pl.ds(wid * _REL_PER_W, _REL_PER_W)], bi_v)
    pltpu.sync_copy(nt_hbm.at[pl.ds(wid * _REL_PER_W, _REL_PER_W)], nt_v)

    def rel_step(i, _):
        b16 = bi_v[pl.ds(i * 16, 16)]
        st16 = plsc.load_gather(st_v, [b16])
        nt16 = nt_v[pl.ds(i * 16, 16)]
        rel_v[pl.ds(i * 16, 16)] = (st16 - nt16).astype(jnp.float32) / 86400.0
        return _

    lax.fori_loop(0, _REL_PER_W // 16, rel_step, None)
    pltpu.sync_copy(rel_v, rel_out.at[pl.ds(wid * _REL_PER_W, _REL_PER_W)])


@functools.cache
def _get_k0():
  return functools.partial(
    pl.kernel,
    mesh=plsc.VectorSubcoreMesh(core_axis_name="c", subcore_axis_name="s"),
    compiler_params=pltpu.CompilerParams(needs_layout_passes=False, use_tc_tiling_on_sc=False),
    out_type=jax.ShapeDtypeStruct((_NP,), jnp.float32),
    scratch_types=[
        pltpu.VMEM((_S,), jnp.int32),
        pltpu.VMEM((_REL_PER_W,), jnp.int32),
        pltpu.VMEM((_REL_PER_W,), jnp.int32),
        pltpu.VMEM((_REL_PER_W,), jnp.float32),
    ],
  )(_k0_body)


_NBUF = 5
_CH = _C // _NC
_NCH2 = _E_PER_SC_TILE // _B
_NGRP = _NCH2 // _NBUF


def _sc2_body(h0v_hbm, e4_hbm, out_hbm, deg_out,
              src2, dst2, rows0, rows1, rows2, rows3, rows4, zbuf,
              deg_v, acc_sh, sem0, sem1, sem2, sem3, sem4):
    cid = lax.axis_index("c")
    sid = lax.axis_index("s")
    rows = [rows0, rows1, rows2, rows3, rows4]
    sems = [sem0, sem1, sem2, sem3, sem4]

    pltpu.sync_copy(e4_hbm.at[0, sid], src2)
    pltpu.sync_copy(e4_hbm.at[1, sid], dst2)

    def _to_slots(chunk):
        for j in range(_B // 16):
            v = src2[chunk, pl.ds(j * 16, 16)]
            src2[chunk, pl.ds(j * 16, 16)] = v + v + cid

    for b in range(_NBUF):
        _to_slots(b)
        pltpu.async_copy(h0v_hbm.at[src2.at[b]], rows[b], sems[b])

    zeros16 = jnp.zeros((16,), jnp.float32)

    def zb_step(r, _):
        for j in range(_CH // 16):
            zbuf[r, pl.ds(j * 16, 16)] = zeros16
        return _

    lax.fori_loop(0, _B, zb_step, None)
    for j in range(_ROWS_PER_TILE // _B):
        pltpu.sync_copy(zbuf, acc_sh.at[pl.ds(sid * _ROWS_PER_TILE + j * _B, _B)])

    def zd_step(i, _):
        deg_v[pl.ds(i * 16, 16)] = zeros16
        return _

    lax.fori_loop(0, _NP // 16, zd_step, None)
    plsc.subcore_barrier()

    ones16 = jnp.full((16,), 1.0, jnp.float32)

    def grp_step(g, _):
        for b in range(_NBUF):
            chunk = g * _NBUF + b
            pltpu.make_async_copy(h0v_hbm.at[src2.at[chunk]], rows[b], sems[b]).wait()
            pltpu.sync_copy(rows[b], acc_sh.at[dst2.at[chunk]], add=True)
            nxt = chunk + _NBUF

            @pl.when(nxt < _NCH2)
            def _refire():
                _to_slots(nxt)
                pltpu.async_copy(h0v_hbm.at[src2.at[nxt]], rows[b], sems[b])

            for j in range(_B // 16):
                d16 = dst2[chunk, pl.ds(j * 16, 16)]
                plsc.addupdate_scatter(deg_v, [d16], ones16)
        return _

    lax.fori_loop(0, _NGRP, grp_step, None)

    @pl.when(cid == 0)
    def _write_deg():
        pltpu.sync_copy(deg_v, deg_out.at[sid])

    plsc.subcore_barrier()

    r0 = sid * _ROWS_PER_TILE

    def wb_step(j, _):
        rr = r0 + j * _B
        pltpu.sync_copy(acc_sh.at[pl.ds(rr, _B)], rows0)
        pltpu.sync_copy(rows0, out_hbm.at[cid, pl.ds(rr, _B)])
        return _

    lax.fori_loop(0, _ROWS_PER_TILE // _B, wb_step, None)


@functools.cache
def _get_sc2():
  return functools.partial(
    pl.kernel,
    mesh=plsc.VectorSubcoreMesh(core_axis_name="c", subcore_axis_name="s"),
    compiler_params=pltpu.CompilerParams(needs_layout_passes=False, use_tc_tiling_on_sc=False),
    out_type=(
        jax.ShapeDtypeStruct((_NC, _NP, _CH), jnp.float32),
        jax.ShapeDtypeStruct((_NS, _NP), jnp.float32),
    ),
    scratch_types=[
        pltpu.VMEM((_NCH2, _B), jnp.int32),
        pltpu.VMEM((_NCH2, _B), jnp.int32),
        pltpu.VMEM((_B, _CH), jnp.float32),
        pltpu.VMEM((_B, _CH), jnp.float32),
        pltpu.VMEM((_B, _CH), jnp.float32),
        pltpu.VMEM((_B, _CH), jnp.float32),
        pltpu.VMEM((_B, _CH), jnp.float32),
        pltpu.VMEM((_B, _CH), jnp.float32),
        pltpu.VMEM((_NP,), jnp.float32),
        pltpu.VMEM_SHARED((_NP, _CH), jnp.float32),
        pltpu.SemaphoreType.DMA,
        pltpu.SemaphoreType.DMA,
        pltpu.SemaphoreType.DMA,
        pltpu.SemaphoreType.DMA,
        pltpu.SemaphoreType.DMA,
    ],
  )(_sc2_body)


def _sc4_body(z_hbm, e4_hbm, hu_hbm, rdeg_hbm, out_hbm,
              z_v, src_v, dst_v, zacc_v, hu_v, rd_v, zslice_v, ztmp_v, o_v,
              zs_sh):
    cid = lax.axis_index("c")
    sid = lax.axis_index("s")

    zeros16 = jnp.zeros((16,), jnp.float32)

    def z_step(i, _):
        zacc_v[pl.ds(i * 16, 16)] = zeros16
        return _

    lax.fori_loop(0, _S // 16, z_step, None)

    pltpu.sync_copy(z_hbm, z_v)
    pltpu.sync_copy(e4_hbm.at[0, sid], src_v)
    pltpu.sync_copy(e4_hbm.at[1, sid], dst_v)

    def e_step(r, _):
        for j in range(_B // 16):
            s16 = src_v[r, pl.ds(j * 16, 16)]
            d16 = dst_v[r, pl.ds(j * 16, 16)]
            zz = plsc.load_gather(z_v, [s16])
            m = d16 < _S
            dsafe = jnp.where(m, d16, 0)
            plsc.addupdate_scatter(zacc_v, [dsafe], zz, mask=m)
        return _

    lax.fori_loop(0, _NCH2, e_step, None)
    pltpu.sync_copy(zacc_v, zs_sh.at[sid])
    plsc.subcore_barrier()

    spw = _S // _NS
    pltpu.sync_copy(zs_sh.at[0, pl.ds(sid * spw, spw)], zslice_v)

    def zc_step(j, _):
        pltpu.sync_copy(zs_sh.at[j, pl.ds(sid * spw, spw)], ztmp_v)

        def za_step(i, __):
            zslice_v[pl.ds(i * 16, 16)] += ztmp_v[pl.ds(i * 16, 16)]
            return __

        lax.fori_loop(0, spw // 16, za_step, None)
        return _

    lax.fori_loop(1, _NS, zc_step, None)
    pltpu.sync_copy(hu_hbm.at[pl.ds(sid * spw, spw)], hu_v)
    pltpu.sync_copy(rdeg_hbm.at[pl.ds(sid * spw, spw)], rd_v)

    def f_step(i, _):
        o_v[pl.ds(i * 16, 16)] = (hu_v[pl.ds(i * 16, 16)]
                                  + zslice_v[pl.ds(i * 16, 16)]
                                  * rd_v[pl.ds(i * 16, 16)])
        return _

    lax.fori_loop(0, spw // 16, f_step, None)

    @pl.when(cid == 0)
    def _write_out():
        pltpu.sync_copy(o_v, out_hbm.at[pl.ds(sid * spw, spw)])


@functools.cache
def _get_sc4():
  return functools.partial(
    pl.kernel,
    mesh=plsc.VectorSubcoreMesh(core_axis_name="c", subcore_axis_name="s"),
    compiler_params=pltpu.CompilerParams(needs_layout_passes=False, use_tc_tiling_on_sc=False),
    out_type=jax.ShapeDtypeStruct((_S,), jnp.float32),
    scratch_types=[
        pltpu.VMEM((_NP,), jnp.float32),
        pltpu.VMEM((_NCH2, _B), jnp.int32),
        pltpu.VMEM((_NCH2, _B), jnp.int32),
        pltpu.VMEM((_S,), jnp.float32),
        pltpu.VMEM((_S // _NS,), jnp.float32),
        pltpu.VMEM((_S // _NS,), jnp.float32),
        pltpu.VMEM((_S // _NS,), jnp.float32),
        pltpu.VMEM((_S // _NS,), jnp.float32),
        pltpu.VMEM((_S // _NS,), jnp.float32),
        pltpu.VMEM_SHARED((_NS, _S), jnp.float32),
    ],
  )(_sc4_body)


_BLK = 2048

_SC_HI = np.float32(1.57080078125)
_SC_MID = np.float32(float(np.pi) / 2.0 - 1.57080078125)


def _sincos(ang):
    t = ang * np.float32(0.6366197723675814)
    n = jnp.floor(t + 0.5)
    ni = n.astype(jnp.int32)
    r = (ang - n * _SC_HI) - n * _SC_MID
    r2 = r * r
    sp = r * (1.0 + r2 * (np.float32(-1.0 / 6) + r2 * (
        np.float32(1.0 / 120) + r2 * np.float32(-1.0 / 5040))))
    cp = 1.0 + r2 * (np.float32(-0.5) + r2 * (np.float32(1.0 / 24) + r2 * (
        np.float32(-1.0 / 720) + r2 * np.float32(1.0 / 40320))))
    q0 = (ni & 1) != 0
    sneg = (ni & 2) != 0
    cneg = ((ni + 1) & 2) != 0
    s = jnp.where(q0, cp, sp)
    c = jnp.where(q0, sp, cp)
    s = jnp.where(sneg, -s, s)
    c = jnp.where(cneg, -c, c)
    return s, c


def _tc1_body(x_ref, rel_ref, div_ref, we_ref, wte_ref, wto_ref, b_ref, o_ref):
    ang = rel_ref[...] * div_ref[...]
    s, c = _sincos(ang)
    h = jnp.dot(x_ref[...], we_ref[...], preferred_element_type=jnp.float32)
    h = h + jnp.dot(s, wte_ref[...], preferred_element_type=jnp.float32)
    h = h + jnp.dot(c, wto_ref[...], preferred_element_type=jnp.float32)
    o_ref[...] = h + b_ref[...]


def _tc1(x_in, rel2, div, W_enc, Wt_e, Wt_o, b01):
    return pl.pallas_call(
        _tc1_body,
        grid=(_NP // _BLK,),
        in_specs=[
            pl.BlockSpec((_BLK, _C), lambda i: (i, 0)),
            pl.BlockSpec((_BLK, 1), lambda i: (i, 0)),
            pl.BlockSpec((1, _C // 2), lambda i: (0, 0)),
            pl.BlockSpec((_C, _C), lambda i: (0, 0)),
            pl.BlockSpec((_C // 2, _C), lambda i: (0, 0)),
            pl.BlockSpec((_C // 2, _C), lambda i: (0, 0)),
            pl.BlockSpec((1, _C), lambda i: (0, 0)),
        ],
        out_specs=pl.BlockSpec((_BLK, _C), lambda i: (i, 0)),
        out_shape=jax.ShapeDtypeStruct((_NP, _C), jnp.float32),
    )(x_in, rel2, div, W_enc, Wt_e, Wt_o, b01)


def _tc3_body(h0_ref, ps_ref, dg_ref, ws_ref, wn_ref, b_ref,
              ws1_ref, wn1_ref, wh_ref, b1_ref, bh_ref, zu_ref, rd_ref):
    ones_c = jnp.ones((_NS, 1), jnp.float32)
    deg_col = lax.dot_general(dg_ref[...], ones_c, (((0,), (0,)), ((), ())),
                              preferred_element_type=jnp.float32)
    rd = 1.0 / jnp.maximum(deg_col, 1.0)
    rd_ref[...] = rd
    aggs = jnp.concatenate([ps_ref[0], ps_ref[1]], axis=1)
    h1 = (jnp.dot(h0_ref[...], ws_ref[...], preferred_element_type=jnp.float32)
          + jnp.dot(aggs, wn_ref[...], preferred_element_type=jnp.float32) * rd
          + b_ref[...])
    h1 = jnp.maximum(h1, 0.0)
    wh = wh_ref[...]
    uv = jnp.concatenate(
        [jnp.dot(wn1_ref[...], wh, preferred_element_type=jnp.float32),
         jnp.dot(ws1_ref[...], wh, preferred_element_type=jnp.float32)], axis=1)
    cval = jnp.dot(b1_ref[...], wh, preferred_element_type=jnp.float32) + bh_ref[...]
    cvec = jnp.concatenate([jnp.zeros((1, 1), jnp.float32), cval], axis=1)
    zu_ref[...] = jnp.dot(h1, uv, preferred_element_type=jnp.float32) + cvec


def _tc3(h0, ps, dg, Ws0, Wn0, b0r, Ws1, Wn1, Wh, b1r, bhr):
    return pl.pallas_call(
        _tc3_body,
        grid=(_NP // _BLK,),
        in_specs=[
            pl.BlockSpec((_BLK, _C), lambda i: (i, 0)),
            pl.BlockSpec((_NC, _BLK, _CH), lambda i: (0, i, 0)),
            pl.BlockSpec((_NS, _BLK), lambda i: (0, i)),
            pl.BlockSpec((_C, _C), lambda i: (0, 0)),
            pl.BlockSpec((_C, _C), lambda i: (0, 0)),
            pl.BlockSpec((1, _C), lambda i: (0, 0)),
            pl.BlockSpec((_C, _C), lambda i: (0, 0)),
            pl.BlockSpec((_C, _C), lambda i: (0, 0)),
            pl.BlockSpec((_C, 1), lambda i: (0, 0)),
            pl.BlockSpec((1, _C), lambda i: (0, 0)),
            pl.BlockSpec((1, 1), lambda i: (0, 0)),
        ],
        out_specs=[
            pl.BlockSpec((_BLK, 2), lambda i: (i, 0)),
            pl.BlockSpec((_BLK, 1), lambda i: (i, 0)),
        ],
        out_shape=[
            jax.ShapeDtypeStruct((_NP, 2), jnp.float32),
            jax.ShapeDtypeStruct((_NP, 1), jnp.float32),
        ],
    )(h0, ps, dg, Ws0, Wn0, b0r, Ws1, Wn1, Wh, b1r, bhr)


def kernel(x, edge_index, node_time, seed_time, batch_idx,
           W_enc, b_enc, Wt, bt, Ws0, Wn0, b0, Ws1, Wn1, b1, Wh, bh):
    nt_p = jnp.pad(node_time, (0, _NP - _N))
    bi_p = jnp.pad(batch_idx, (0, _NP - _N))
    src = edge_index[0]
    dst = edge_index[1]

    Wt_e = Wt[0::2]
    Wt_o = Wt[1::2]
    b01 = (b_enc + bt).reshape(1, _C)
    div = jnp.asarray(
        np.exp(-np.arange(0, _C, 2, dtype=np.float64) * (np.log(10000.0) / _C)),
        jnp.float32).reshape(1, _C // 2)

    rel_t = _get_k0()(nt_p, seed_time, bi_p)
    h0 = _tc1(x, rel_t.reshape(_NP, 1), div, W_enc, Wt_e, Wt_o, b01)
    h0v = h0.reshape(2 * _NP, _CH)
    e4 = edge_index.reshape(2, _NS, _NCH2, _B)
    parts, degp = _get_sc2()(h0v, e4)
    zu, rd = _tc3(h0, parts, degp, Ws0, Wn0, b0.reshape(1, _C),
                  Ws1, Wn1, Wh, b1.reshape(1, _C), bh.reshape(1, 1))
    z = zu[:, 0]
    hu = zu[:_S, 1]
    rd1k = rd[:_S, 0]
    out = _get_sc4()(z, e4, hu, rd1k)
    return out.reshape(_S, 1)


# speedup vs baseline: 22.0167x; 1.0626x over previous
import functools

import numpy as np
import jax
import jax.numpy as jnp
from jax import lax
from jax.experimental import pallas as pl
from jax.experimental.pallas import tpu as pltpu
from jax.experimental.pallas import tpu_sc as plsc

_N = 10000
_E = 320000
_C = 128
_S = 1024
_NP = 10240

_NC = 2
_NS = 16
_NW = _NC * _NS

_REL_PER_W = _NP // _NW
_E_PER_SC_TILE = _E // _NS
_E_PER_W = _E // _NW
_B = 80
_NCHUNK = _E_PER_W // _B
_ROWS_PER_TILE = _NP // _NS

def _k0_body(nt_hbm, st_hbm, bi_hbm, rel_out, st_v, bi_v, nt_v, rel_v):
    cid = lax.axis_index("c")
    sid = lax.axis_index("s")
    wid = sid * _NC + cid

    pltpu.sync_copy(st_hbm, st_v)
    pltpu.sync_copy(bi_hbm.at[pl.ds(wid * _REL_PER_W, _REL_PER_W)], bi_v)
    pltpu.sync_copy(nt_hbm.at[pl.ds(wid * _REL_PER_W, _REL_PER_W)], nt_v)

    def rel_step(i, _):
        b16 = bi_v[pl.ds(i * 16, 16)]
        st16 = plsc.load_gather(st_v, [b16])
        nt16 = nt_v[pl.ds(i * 16, 16)]
        rel_v[pl.ds(i * 16, 16)] = (st16 - nt16).astype(jnp.float32) / 86400.0
        return _

    lax.fori_loop(0, _REL_PER_W // 16, rel_step, None)
    pltpu.sync_copy(rel_v, rel_out.at[pl.ds(wid * _REL_PER_W, _REL_PER_W)])


@functools.cache
def _get_k0():
  return functools.partial(
    pl.kernel,
    mesh=plsc.VectorSubcoreMesh(core_axis_name="c", subcore_axis_name="s"),
    compiler_params=pltpu.CompilerParams(needs_layout_passes=False, use_tc_tiling_on_sc=False),
    out_type=jax.ShapeDtypeStruct((_NP,), jnp.float32),
    scratch_types=[
        pltpu.VMEM((_S,), jnp.int32),
        pltpu.VMEM((_REL_PER_W,), jnp.int32),
        pltpu.VMEM((_REL_PER_W,), jnp.int32),
        pltpu.VMEM((_REL_PER_W,), jnp.float32),
    ],
  )(_k0_body)


_NBUF = 5
_CH = _C // _NC
_NCH2 = _E_PER_SC_TILE // _B
_NGRP = _NCH2 // _NBUF


def _sc2_body(h0v_hbm, e4_hbm, out_hbm, deg_out,
              src2, dst2, rows0, rows1, rows2, rows3, rows4, zbuf,
              deg_v, acc_sh, sem0, sem1, sem2, sem3, sem4):
    cid = lax.axis_index("c")
    sid = lax.axis_index("s")
    rows = [rows0, rows1, rows2, rows3, rows4]
    sems = [sem0, sem1, sem2, sem3, sem4]

    pltpu.sync_copy(e4_hbm.at[0, sid], src2)
    pltpu.sync_copy(e4_hbm.at[1, sid], dst2)

    def _to_slots(chunk):
        for j in range(_B // 16):
            v = src2[chunk, pl.ds(j * 16, 16)]
            src2[chunk, pl.ds(j * 16, 16)] = v + v + cid

    for b in range(_NBUF):
        _to_slots(b)
        pltpu.async_copy(h0v_hbm.at[src2.at[b]], rows[b], sems[b])

    zeros16 = jnp.zeros((16,), jnp.float32)

    def zb_step(r, _):
        for j in range(_CH // 16):
            zbuf[r, pl.ds(j * 16, 16)] = zeros16
        return _

    lax.fori_loop(0, _B, zb_step, None)
    for j in range(_ROWS_PER_TILE // _B):
        pltpu.sync_copy(zbuf, acc_sh.at[pl.ds(sid * _ROWS_PER_TILE + j * _B, _B)])

    def zd_step(i, _):
        deg_v[pl.ds(i * 16, 16)] = zeros16
        return _

    lax.fori_loop(0, _NP // 16, zd_step, None)
    plsc.subcore_barrier()

    ones16 = jnp.full((16,), 1.0, jnp.float32)

    def grp_step(g, _):
        for b in range(_NBUF):
            chunk = g * _NBUF + b
            pltpu.make_async_copy(h0v_hbm.at[src2.at[chunk]], rows[b], sems[b]).wait()
            pltpu.sync_copy(rows[b], acc_sh.at[dst2.at[chunk]], add=True)
            nxt = chunk + _NBUF

            @pl.when(nxt < _NCH2)
            def _refire():
                _to_slots(nxt)
                pltpu.async_copy(h0v_hbm.at[src2.at[nxt]], rows[b], sems[b])

            for j in range(_B // 16):
                d16 = dst2[chunk, pl.ds(j * 16, 16)]
                plsc.addupdate_scatter(deg_v, [d16], ones16)
        return _

    lax.fori_loop(0, _NGRP, grp_step, None)

    @pl.when(cid == 0)
    def _write_deg():
        pltpu.sync_copy(deg_v, deg_out.at[sid])

    plsc.subcore_barrier()

    r0 = sid * _ROWS_PER_TILE

    def wb_step(j, _):
        rr = r0 + j * _B
        pltpu.sync_copy(acc_sh.at[pl.ds(rr, _B)], rows0)
        pltpu.sync_copy(rows0, out_hbm.at[pl.ds(rr, _B), pl.ds(cid * _CH, _CH)])
        return _

    lax.fori_loop(0, _ROWS_PER_TILE // _B, wb_step, None)


@functools.cache
def _get_sc2():
  return functools.partial(
    pl.kernel,
    mesh=plsc.VectorSubcoreMesh(core_axis_name="c", subcore_axis_name="s"),
    compiler_params=pltpu.CompilerParams(needs_layout_passes=False, use_tc_tiling_on_sc=False),
    out_type=(
        jax.ShapeDtypeStruct((_NP, _C), jnp.float32),
        jax.ShapeDtypeStruct((_NS, _NP), jnp.float32),
    ),
    scratch_types=[
        pltpu.VMEM((_NCH2, _B), jnp.int32),
        pltpu.VMEM((_NCH2, _B), jnp.int32),
        pltpu.VMEM((_B, _CH), jnp.float32),
        pltpu.VMEM((_B, _CH), jnp.float32),
        pltpu.VMEM((_B, _CH), jnp.float32),
        pltpu.VMEM((_B, _CH), jnp.float32),
        pltpu.VMEM((_B, _CH), jnp.float32),
        pltpu.VMEM((_B, _CH), jnp.float32),
        pltpu.VMEM((_NP,), jnp.float32),
        pltpu.VMEM_SHARED((_NP, _CH), jnp.float32),
        pltpu.SemaphoreType.DMA,
        pltpu.SemaphoreType.DMA,
        pltpu.SemaphoreType.DMA,
        pltpu.SemaphoreType.DMA,
        pltpu.SemaphoreType.DMA,
    ],
  )(_sc2_body)


def _sc4_body(z_hbm, e4_hbm, hu_hbm, rdeg_hbm, out_hbm,
              z_v, src_v, dst_v, zacc_v, hu_v, rd_v, zslice_v, ztmp_v, o_v,
              zs_sh):
    cid = lax.axis_index("c")
    sid = lax.axis_index("s")

    zeros16 = jnp.zeros((16,), jnp.float32)

    def z_step(i, _):
        zacc_v[pl.ds(i * 16, 16)] = zeros16
        return _

    lax.fori_loop(0, _S // 16, z_step, None)

    pltpu.sync_copy(z_hbm, z_v)
    pltpu.sync_copy(e4_hbm.at[0, sid], src_v)
    pltpu.sync_copy(e4_hbm.at[1, sid], dst_v)

    def e_step(r, _):
        for j in range(_B // 16):
            s16 = src_v[r, pl.ds(j * 16, 16)]
            d16 = dst_v[r, pl.ds(j * 16, 16)]
            zz = plsc.load_gather(z_v, [s16])
            m = d16 < _S
            dsafe = jnp.where(m, d16, 0)
            plsc.addupdate_scatter(zacc_v, [dsafe], zz, mask=m)
        return _

    lax.fori_loop(0, _NCH2, e_step, None)
    pltpu.sync_copy(zacc_v, zs_sh.at[sid])
    plsc.subcore_barrier()

    spw = _S // _NS
    pltpu.sync_copy(zs_sh.at[0, pl.ds(sid * spw, spw)], zslice_v)

    def zc_step(j, _):
        pltpu.sync_copy(zs_sh.at[j, pl.ds(sid * spw, spw)], ztmp_v)

        def za_step(i, __):
            zslice_v[pl.ds(i * 16, 16)] += ztmp_v[pl.ds(i * 16, 16)]
            return __

        lax.fori_loop(0, spw // 16, za_step, None)
        return _

    lax.fori_loop(1, _NS, zc_step, None)
    pltpu.sync_copy(hu_hbm.at[pl.ds(sid * spw, spw)], hu_v)
    pltpu.sync_copy(rdeg_hbm.at[pl.ds(sid * spw, spw)], rd_v)

    def f_step(i, _):
        o_v[pl.ds(i * 16, 16)] = (hu_v[pl.ds(i * 16, 16)]
                                  + zslice_v[pl.ds(i * 16, 16)]
                                  * rd_v[pl.ds(i * 16, 16)])
        return _

    lax.fori_loop(0, spw // 16, f_step, None)

    @pl.when(cid == 0)
    def _write_out():
        pltpu.sync_copy(o_v, out_hbm.at[pl.ds(sid * spw, spw)])


@functools.cache
def _get_sc4():
  return functools.partial(
    pl.kernel,
    mesh=plsc.VectorSubcoreMesh(core_axis_name="c", subcore_axis_name="s"),
    compiler_params=pltpu.CompilerParams(needs_layout_passes=False, use_tc_tiling_on_sc=False),
    out_type=jax.ShapeDtypeStruct((_S,), jnp.float32),
    scratch_types=[
        pltpu.VMEM((_NP,), jnp.float32),
        pltpu.VMEM((_NCH2, _B), jnp.int32),
        pltpu.VMEM((_NCH2, _B), jnp.int32),
        pltpu.VMEM((_S,), jnp.float32),
        pltpu.VMEM((_S // _NS,), jnp.float32),
        pltpu.VMEM((_S // _NS,), jnp.float32),
        pltpu.VMEM((_S // _NS,), jnp.float32),
        pltpu.VMEM((_S // _NS,), jnp.float32),
        pltpu.VMEM((_S // _NS,), jnp.float32),
        pltpu.VMEM_SHARED((_NS, _S), jnp.float32),
    ],
  )(_sc4_body)


_BLK = 2048

_SC_HI = np.float32(1.57080078125)
_SC_MID = np.float32(float(np.pi) / 2.0 - 1.57080078125)


def _sincos(ang):
    t = ang * np.float32(0.6366197723675814)
    n = jnp.floor(t + 0.5)
    ni = n.astype(jnp.int32)
    r = (ang - n * _SC_HI) - n * _SC_MID
    r2 = r * r
    sp = r * (1.0 + r2 * (np.float32(-1.0 / 6) + r2 * (
        np.float32(1.0 / 120) + r2 * np.float32(-1.0 / 5040))))
    cp = 1.0 + r2 * (np.float32(-0.5) + r2 * (np.float32(1.0 / 24) + r2 * (
        np.float32(-1.0 / 720) + r2 * np.float32(1.0 / 40320))))
    q0 = (ni & 1) != 0
    sneg = (ni & 2) != 0
    cneg = ((ni + 1) & 2) != 0
    s = jnp.where(q0, cp, sp)
    c = jnp.where(q0, sp, cp)
    s = jnp.where(sneg, -s, s)
    c = jnp.where(cneg, -c, c)
    return s, c


def _tc1_body(x_ref, rel_ref, div_ref, we_ref, wte_ref, wto_ref, b_ref, o_ref):
    ang = rel_ref[...] * div_ref[...]
    s, c = _sincos(ang)
    h = jnp.dot(x_ref[...], we_ref[...], preferred_element_type=jnp.float32)
    h = h + jnp.dot(s, wte_ref[...], preferred_element_type=jnp.float32)
    h = h + jnp.dot(c, wto_ref[...], preferred_element_type=jnp.float32)
    o_ref[...] = h + b_ref[...]


def _tc1(x_in, rel2, div, W_enc, Wt_e, Wt_o, b01):
    return pl.pallas_call(
        _tc1_body,
        grid=(_NP // _BLK,),
        in_specs=[
            pl.BlockSpec((_BLK, _C), lambda i: (i, 0)),
            pl.BlockSpec((_BLK, 1), lambda i: (i, 0)),
            pl.BlockSpec((1, _C // 2), lambda i: (0, 0)),
            pl.BlockSpec((_C, _C), lambda i: (0, 0)),
            pl.BlockSpec((_C // 2, _C), lambda i: (0, 0)),
            pl.BlockSpec((_C // 2, _C), lambda i: (0, 0)),
            pl.BlockSpec((1, _C), lambda i: (0, 0)),
        ],
        out_specs=pl.BlockSpec((_BLK, _C), lambda i: (i, 0)),
        out_shape=jax.ShapeDtypeStruct((_NP, _C), jnp.float32),
    )(x_in, rel2, div, W_enc, Wt_e, Wt_o, b01)


def _tc3_body(h0_ref, ps_ref, dg_ref, ws_ref, wn_ref, b_ref,
              ws1_ref, wn1_ref, wh_ref, b1_ref, bh_ref, zu_ref, rd_ref):
    ones_c = jnp.ones((_NS, 1), jnp.float32)
    deg_col = lax.dot_general(dg_ref[...], ones_c, (((0,), (0,)), ((), ())),
                              preferred_element_type=jnp.float32)
    rd = 1.0 / jnp.maximum(deg_col, 1.0)
    rd_ref[...] = rd
    aggs = ps_ref[...]
    h1 = (jnp.dot(h0_ref[...], ws_ref[...], preferred_element_type=jnp.float32)
          + jnp.dot(aggs, wn_ref[...], preferred_element_type=jnp.float32) * rd
          + b_ref[...])
    h1 = jnp.maximum(h1, 0.0)
    wh = wh_ref[...]
    uv = jnp.concatenate(
        [jnp.dot(wn1_ref[...], wh, preferred_element_type=jnp.float32),
         jnp.dot(ws1_ref[...], wh, preferred_element_type=jnp.float32)], axis=1)
    cval = jnp.dot(b1_ref[...], wh, preferred_element_type=jnp.float32) + bh_ref[...]
    cvec = jnp.concatenate([jnp.zeros((1, 1), jnp.float32), cval], axis=1)
    zu_ref[...] = jnp.dot(h1, uv, preferred_element_type=jnp.float32) + cvec


def _tc3(h0, ps, dg, Ws0, Wn0, b0r, Ws1, Wn1, Wh, b1r, bhr):
    return pl.pallas_call(
        _tc3_body,
        grid=(_NP // _BLK,),
        in_specs=[
            pl.BlockSpec((_BLK, _C), lambda i: (i, 0)),
            pl.BlockSpec((_BLK, _C), lambda i: (i, 0)),
            pl.BlockSpec((_NS, _BLK), lambda i: (0, i)),
            pl.BlockSpec((_C, _C), lambda i: (0, 0)),
            pl.BlockSpec((_C, _C), lambda i: (0, 0)),
            pl.BlockSpec((1, _C), lambda i: (0, 0)),
            pl.BlockSpec((_C, _C), lambda i: (0, 0)),
            pl.BlockSpec((_C, _C), lambda i: (0, 0)),
            pl.BlockSpec((_C, 1), lambda i: (0, 0)),
            pl.BlockSpec((1, _C), lambda i: (0, 0)),
            pl.BlockSpec((1, 1), lambda i: (0, 0)),
        ],
        out_specs=[
            pl.BlockSpec((_BLK, 2), lambda i: (i, 0)),
            pl.BlockSpec((_BLK, 1), lambda i: (i, 0)),
        ],
        out_shape=[
            jax.ShapeDtypeStruct((_NP, 2), jnp.float32),
            jax.ShapeDtypeStruct((_NP, 1), jnp.float32),
        ],
    )(h0, ps, dg, Ws0, Wn0, b0r, Ws1, Wn1, Wh, b1r, bhr)


def kernel(x, edge_index, node_time, seed_time, batch_idx,
           W_enc, b_enc, Wt, bt, Ws0, Wn0, b0, Ws1, Wn1, b1, Wh, bh):
    nt_p = jnp.pad(node_time, (0, _NP - _N))
    bi_p = jnp.pad(batch_idx, (0, _NP - _N))
    src = edge_index[0]
    dst = edge_index[1]

    Wt_e = Wt[0::2]
    Wt_o = Wt[1::2]
    b01 = (b_enc + bt).reshape(1, _C)
    div = jnp.asarray(
        np.exp(-np.arange(0, _C, 2, dtype=np.float64) * (np.log(10000.0) / _C)),
        jnp.float32).reshape(1, _C // 2)

    rel_t = _get_k0()(nt_p, seed_time, bi_p)
    h0 = _tc1(x, rel_t.reshape(_NP, 1), div, W_enc, Wt_e, Wt_o, b01)
    h0v = h0.reshape(2 * _NP, _CH)
    e4 = edge_index.reshape(2, _NS, _NCH2, _B)
    parts, degp = _get_sc2()(h0v, e4)
    zu, rd = _tc3(h0, parts, degp, Ws0, Wn0, b0.reshape(1, _C),
                  Ws1, Wn1, Wh, b1.reshape(1, _C), bh.reshape(1, 1))
    z = zu[:, 0]
    hu = zu[:_S, 1]
    rd1k = rd[:_S, 0]
    out = _get_sc4()(z, e4, hu, rd1k)
    return out.reshape(_S, 1)
